# Initial kernel scaffold; baseline (speedup 1.0000x reference)
import jax
import jax.numpy as jnp
from jax.experimental import pallas as pl


def kernel(x, edge_index, W1l, W1r, att1, b1, W3l, W3r, att3, b3):
    raise NotImplementedError("write your pallas kernel here")


# speedup vs baseline: 10.8109x; 10.8109x over previous
import functools

import jax
import jax.numpy as jnp
from jax import lax
from jax.experimental import pallas as pl
from jax.experimental.pallas import tpu as pltpu
from jax.experimental.pallas import tpu_sc as plsc

N = 10000
E = 320000
IN_DIM = 128
HID = 16
HEADS = 8
OUT_DIM = 64

NC = 2
NS = 16
NW = NC * NS
EPW = E // NW
KE = 80
NBLK = EPW // KE
NACC = 10240
DROWS = 640
RCH = 8


def _make_edge_pass(D, H, xroff):
    VPH = D // 16 // H
    NROWS = NACC + (DROWS if H > 1 else 0)
    RPT = NROWS // NS
    NCH = RPT // RCH

    mesh = plsc.VectorSubcoreMesh(core_axis_name="c", subcore_axis_name="s")

    scratch = [
        pltpu.VMEM((KE,), jnp.int32),
        pltpu.VMEM((KE,), jnp.int32),
        pltpu.VMEM((KE, 128), jnp.float32),
        pltpu.VMEM((KE, 128), jnp.float32),
        pltpu.VMEM((KE, 128), jnp.float32),
        pltpu.VMEM((D,), jnp.float32),
        pltpu.VMEM((RCH, 128), jnp.float32),
        pltpu.VMEM_SHARED((NROWS, 128), jnp.float32),
        pltpu.SemaphoreType.DMA,
        pltpu.SemaphoreType.DMA,
    ]
    if H > 1:
        scratch += [
            pltpu.VMEM((KE,), jnp.int32),
            pltpu.VMEM((KE, 128), jnp.float32),
        ]

    @functools.partial(
        pl.kernel,
        mesh=mesh,
        out_type=jax.ShapeDtypeStruct((NC, NROWS, 128), jnp.float32),
        scratch_types=scratch,
    )
    def edge_pass(xl_hbm, xr_hbm, eidx_hbm, att_hbm, out_hbm,
                  srcv, dstv, xlr, xrr, wv, attv, rowbuf,
                  acc, sem1, sem2, *den_scratch):
        if H > 1:
            didxv, wd = den_scratch
        c = lax.axis_index("c")
        s = lax.axis_index("s")
        wid = c * NS + s

        pltpu.sync_copy(att_hbm, attv)

        zerov = jnp.zeros((16,), jnp.float32)

        def _zrow(i, carry):
            for j in range(8):
                rowbuf[i, pl.ds(j * 16, 16)] = zerov
            return carry
        lax.fori_loop(0, RCH, _zrow, 0)
        for k in range(NCH):
            r0 = s * RPT + k * RCH
            pltpu.sync_copy(rowbuf, acc.at[pl.ds(r0, RCH)])
        plsc.subcore_barrier()

        if H == 1:
            def _ztail(e, carry):
                for j in range(5, 8):
                    wv[e, pl.ds(j * 16, 16)] = zerov
                return carry
            lax.fori_loop(0, KE, _ztail, 0)

        lane = lax.iota(jnp.int32, 16)
        gdn = lax.GatherDimensionNumbers(
            offset_dims=(), collapsed_slice_dims=(0,), start_index_map=(0,))

        def _perm(u, xor):
            return lax.gather(u, (lane ^ xor)[:, None], gdn, (1,),
                              mode=lax.GatherScatterMode.PROMISE_IN_BOUNDS)

        def _hsum(u):
            for k2 in (8, 4, 2, 1):
                u = u + _perm(u, k2)
            return u

        def _egroup(g, carry):
            if H > 1:
                dgrp = dstv[pl.ds(g * 16, 16)]
                og = dgrp & 15
                oddfv = (og & 1).astype(jnp.float32)
                vvtfv = lax.shift_right_logical(og, 1).astype(jnp.float32)
            for j in range(16):
                e = g * 16 + j
                exbs = []
                for h in range(H):
                    usum = None
                    xls = []
                    for jj in range(VPH):
                        v = h * VPH + jj
                        xlv = xlr[e, pl.ds(v * 16, 16)]
                        xrv = xrr[e, pl.ds(xroff + v * 16, 16)]
                        t = xlv + xrv
                        t = jnp.where(t >= 0.0, t, t * 0.2)
                        u = t * attv[pl.ds(v * 16, 16)]
                        usum = u if usum is None else usum + u
                        xls.append(xlv)
                    exb = jnp.exp(_hsum(usum))
                    exbs.append(exb)
                    for jj in range(VPH):
                        v = h * VPH + jj
                        wv[e, pl.ds(v * 16, 16)] = xls[jj] * exb
                if H == 1:
                    wv[e, pl.ds(D, 16)] = exbs[0]
                else:
                    exrow = zerov
                    for h in range(H):
                        exrow = jnp.where(lane == h, exbs[h], exrow)
                    oddf = oddfv[j]
                    vvtf = vvtfv[j]
                    shifted = exrow + oddf * (_perm(exrow, 8) - exrow)
                    for vv in range(8):
                        m = jnp.maximum(
                            0.0, 1.0 - jnp.abs(vvtf - float(vv)))
                        wd[e, pl.ds(vv * 16, 16)] = shifted * m
            return carry

        def _blk(b, carry):
            base = wid * EPW + b * KE
            pltpu.sync_copy(eidx_hbm.at[pl.ds(base, KE)], srcv)
            pltpu.sync_copy(eidx_hbm.at[pl.ds(E + base, KE)], dstv)
            cp1 = pltpu.async_copy(xl_hbm.at[srcv], xlr, sem1)
            cp2 = pltpu.async_copy(xr_hbm.at[dstv], xrr, sem2)
            cp1.wait()
            cp2.wait()
            if H > 1:
                for g in range(KE // 16):
                    dv = dstv[pl.ds(g * 16, 16)]
                    didxv[pl.ds(g * 16, 16)] = (
                        lax.shift_right_logical(dv, 4) + NACC)
            lax.fori_loop(0, KE // 16, _egroup, 0)
            pltpu.sync_copy(wv, acc.at[dstv], add=True)
            if H > 1:
                pltpu.sync_copy(wd, acc.at[didxv], add=True)
            return carry

        lax.fori_loop(0, NBLK, _blk, 0)
        plsc.subcore_barrier()

        for k in range(NCH):
            r0 = s * RPT + k * RCH
            pltpu.sync_copy(acc.at[pl.ds(r0, RCH)], rowbuf)
            pltpu.sync_copy(rowbuf, out_hbm.at[c, pl.ds(r0, RCH)])

    return edge_pass


_edge_pass_1 = _make_edge_pass(HEADS * HID, HEADS, 0)
_edge_pass_2 = _make_edge_pass(OUT_DIM, 1, OUT_DIM)

_RB = 400
_NG = N // _RB
_RBA = 512
_NGA = NACC // _RBA


def _tc_in_proj(x, Wl, Wr):
    def body(x_ref, wl_ref, wr_ref, xl_ref, xr_ref):
        xb = x_ref[...]
        xl_ref[...] = jnp.dot(xb, wl_ref[...], preferred_element_type=jnp.float32)
        xr_ref[...] = jnp.dot(xb, wr_ref[...], preferred_element_type=jnp.float32)

    return pl.pallas_call(
        body,
        grid=(_NG,),
        in_specs=[
            pl.BlockSpec((_RB, IN_DIM), lambda i: (i, 0)),
            pl.BlockSpec((IN_DIM, IN_DIM), lambda i: (0, 0)),
            pl.BlockSpec((IN_DIM, IN_DIM), lambda i: (0, 0)),
        ],
        out_specs=[
            pl.BlockSpec((_RB, IN_DIM), lambda i: (i, 0)),
            pl.BlockSpec((_RB, IN_DIM), lambda i: (i, 0)),
        ],
        out_shape=[
            jax.ShapeDtypeStruct((N, IN_DIM), jnp.float32),
            jax.ShapeDtypeStruct((N, IN_DIM), jnp.float32),
        ],
    )(x, Wl, Wr)


def _tc_mid(nd, den8, b1, W3l, W3r):
    D1 = HEADS * HID

    def body(num_ref, den_ref, b1_ref, wl_ref, wr_ref, o_ref):
        num = num_ref[0] + num_ref[1]
        den = den_ref[0] + den_ref[1]
        col = lax.broadcasted_iota(jnp.int32, (HEADS, D1), 1)
        row = lax.broadcasted_iota(jnp.int32, (HEADS, D1), 0)
        bmat = jnp.where(col // HID == row, 1.0, 0.0)
        den_b = jnp.dot(den, bmat, preferred_element_type=jnp.float32)
        h = num / (den_b + 1e-16) + b1_ref[...]
        h = jnp.where(h > 0.0, h, jnp.exp(h) - 1.0)
        o_ref[:, :OUT_DIM] = jnp.dot(
            h, wl_ref[...], preferred_element_type=jnp.float32)
        o_ref[:, OUT_DIM:] = jnp.dot(
            h, wr_ref[...], preferred_element_type=jnp.float32)

    return pl.pallas_call(
        body,
        grid=(_NGA,),
        in_specs=[
            pl.BlockSpec((NC, _RBA, 128), lambda i: (0, i, 0)),
            pl.BlockSpec((NC, _RBA, HEADS), lambda i: (0, i, 0)),
            pl.BlockSpec((1, D1), lambda i: (0, 0)),
            pl.BlockSpec((D1, OUT_DIM), lambda i: (0, 0)),
            pl.BlockSpec((D1, OUT_DIM), lambda i: (0, 0)),
        ],
        out_specs=pl.BlockSpec((_RBA, 2 * OUT_DIM), lambda i: (i, 0)),
        out_shape=jax.ShapeDtypeStruct((NACC, 2 * OUT_DIM), jnp.float32),
    )(nd, den8, b1, W3l, W3r)


def _tc_final(nd, b3):
    def body(nd_ref, b3_ref, o_ref, ls_ref):
        nd0 = nd_ref[0]
        ndb = nd_ref[1]
        num = nd0[:, :OUT_DIM] + ndb[:, :OUT_DIM]
        den = nd0[:, OUT_DIM:OUT_DIM + 1] + ndb[:, OUT_DIM:OUT_DIM + 1]
        o = num / (den + 1e-16) + b3_ref[...]
        o = jnp.where(o > 0.0, o, jnp.exp(o) - 1.0)
        m = jnp.max(o, axis=1, keepdims=True)
        lse = m + jnp.log(jnp.sum(jnp.exp(o - m), axis=1, keepdims=True))
        o_ref[...] = o
        ls_ref[...] = o - lse

    return pl.pallas_call(
        body,
        grid=(_NGA,),
        in_specs=[
            pl.BlockSpec((NC, _RBA, 128), lambda i: (0, i, 0)),
            pl.BlockSpec((1, OUT_DIM), lambda i: (0, 0)),
        ],
        out_specs=[
            pl.BlockSpec((_RBA, OUT_DIM), lambda i: (i, 0)),
            pl.BlockSpec((_RBA, OUT_DIM), lambda i: (i, 0)),
        ],
        out_shape=[
            jax.ShapeDtypeStruct((NACC, OUT_DIM), jnp.float32),
            jax.ShapeDtypeStruct((NACC, OUT_DIM), jnp.float32),
        ],
    )(nd, b3)


def kernel(x, edge_index, W1l, W1r, att1, b1, W3l, W3r, att3, b3):
    eidx = jnp.reshape(edge_index, (-1,))
    xl1, xr1 = _tc_in_proj(x, W1l, W1r)
    nd1 = _edge_pass_1(xl1, xr1, eidx, jnp.reshape(att1, (-1,)))
    den8 = jnp.reshape(nd1[:, NACC:, :], (NC, NACC, HEADS))
    hcat = _tc_mid(nd1, den8, jnp.reshape(b1, (1, -1)), W3l, W3r)
    nd2 = _edge_pass_2(hcat, hcat, eidx, jnp.reshape(att3, (-1,)))
    o, ls = _tc_final(nd2, jnp.reshape(b3, (1, -1)))
    return (o[:N], ls[:N])


# speedup vs baseline: 23.6690x; 2.1894x over previous
import functools

import jax
import jax.numpy as jnp
from jax import lax
from jax.experimental import pallas as pl
from jax.experimental.pallas import tpu as pltpu
from jax.experimental.pallas import tpu_sc as plsc

N = 10000
E = 320000
IN_DIM = 128
HID = 16
HEADS = 8
OUT_DIM = 64

NC = 2
NS = 16
NW = NC * NS
EPW = E // NW
KE = 40
KD = 48
NBLK = EPW // KE
NACC = 10240
DROWS = 640
RCH = 8


def _make_edge_pass(D, H, xroff):
    VPH = D // 16 // H
    NROWS = NACC + (DROWS if H > 1 else 0)
    RPT = NROWS // NS
    NCH = RPT // RCH

    mesh = plsc.VectorSubcoreMesh(core_axis_name="c", subcore_axis_name="s")

    scratch = [
        pltpu.VMEM((KE,), jnp.int32),
        pltpu.VMEM((KE,), jnp.int32),
        pltpu.VMEM((KD,), jnp.int32),
        pltpu.VMEM((KD,), jnp.int32),
        pltpu.VMEM((KE, 128), jnp.float32),
        pltpu.VMEM((KE, 128), jnp.float32),
        pltpu.VMEM((KD, 128), jnp.float32),
        pltpu.VMEM((KD, 128), jnp.float32),
        pltpu.VMEM((KD, 128), jnp.float32),
        pltpu.VMEM((D,), jnp.float32),
        pltpu.VMEM((RCH, 128), jnp.float32),
        pltpu.VMEM_SHARED((NROWS, 128), jnp.float32),
        pltpu.SemaphoreType.DMA,
        pltpu.SemaphoreType.DMA,
        pltpu.SemaphoreType.DMA,
        pltpu.SemaphoreType.DMA,
        pltpu.SemaphoreType.DMA,
        pltpu.SemaphoreType.DMA,
        pltpu.SemaphoreType.DMA,
        pltpu.SemaphoreType.DMA,
    ]
    if H > 1:
        scratch += [
            pltpu.VMEM((KD,), jnp.int32),
            pltpu.VMEM((KD, 128), jnp.float32),
        ]

    @functools.partial(
        pl.kernel,
        mesh=mesh,
        out_type=jax.ShapeDtypeStruct((NC, NROWS, 128), jnp.float32),
        scratch_types=scratch,
    )
    def edge_pass(xl_hbm, xr_hbm, eidx_hbm, att_hbm, out_hbm,
                  s0, s1, d0, d1, gl0, gl1, gr0, gr1, wv, attv, rowbuf,
                  acc, ss0, ss1, sd0, sd1, sgl0, sgl1, sgr0, sgr1,
                  *den_scratch):
        if H > 1:
            didxv, wd = den_scratch
        S = (s0, s1)
        DD = (d0, d1)
        GL = (gl0, gl1)
        GR = (gr0, gr1)
        SS = (ss0, ss1)
        SD = (sd0, sd1)
        SGL = (sgl0, sgl1)
        SGR = (sgr0, sgr1)
        c = lax.axis_index("c")
        s = lax.axis_index("s")
        wid = c * NS + s

        pltpu.sync_copy(att_hbm, attv)

        zerov = jnp.zeros((16,), jnp.float32)

        def _zrow(i, carry):
            for j in range(8):
                rowbuf[i, pl.ds(j * 16, 16)] = zerov
            return carry
        lax.fori_loop(0, RCH, _zrow, 0)
        for k in range(NCH):
            r0 = s * RPT + k * RCH
            pltpu.sync_copy(rowbuf, acc.at[pl.ds(r0, RCH)])
        plsc.subcore_barrier()

        if H == 1:
            def _ztail(e, carry):
                for j in range(5, 8):
                    wv[e, pl.ds(j * 16, 16)] = zerov
                return carry
            lax.fori_loop(0, KD, _ztail, 0)
        for e in range(KE, KD):
            for j in range(8):
                wv[e, pl.ds(j * 16, 16)] = zerov
                if H > 1:
                    wd[e, pl.ds(j * 16, 16)] = zerov

        attvs = [attv[pl.ds(v * 16, 16)] for v in range(D // 16)]
        lane = lax.iota(jnp.int32, 16)
        gdn = lax.GatherDimensionNumbers(
            offset_dims=(), collapsed_slice_dims=(0,), start_index_map=(0,))

        def _perm(u, xor):
            return lax.gather(u, (lane ^ xor)[:, None], gdn, (1,),
                              mode=lax.GatherScatterMode.PROMISE_IN_BOUNDS)

        def _hsum(u):
            for k2 in (8, 4, 2, 1):
                u = u + _perm(u, k2)
            return u

        def _compute_block(xlr, xrr, dstv):
            for g, jmax in ((0, 16), (1, 16), (2, 8)):
                if H > 1:
                    dgrp = dstv[pl.ds(g * 16, 16)]
                    og = dgrp & 15
                    oddfv = (og & 1).astype(jnp.float32)
                    vvtfv = lax.shift_right_logical(og, 1).astype(
                        jnp.float32)
                for j in range(jmax):
                    e = g * 16 + j
                    exbs = []
                    for h in range(H):
                        usum = None
                        xls = []
                        for jj in range(VPH):
                            v = h * VPH + jj
                            xlv = xlr[e, pl.ds(v * 16, 16)]
                            xrv = xrr[e, pl.ds(xroff + v * 16, 16)]
                            t = xlv + xrv
                            t = jnp.where(t >= 0.0, t, t * 0.2)
                            u = t * attvs[v]
                            usum = u if usum is None else usum + u
                            xls.append(xlv)
                        exb = jnp.exp(_hsum(usum))
                        exbs.append(exb)
                        for jj in range(VPH):
                            v = h * VPH + jj
                            wv[e, pl.ds(v * 16, 16)] = xls[jj] * exb
                    if H == 1:
                        wv[e, pl.ds(D, 16)] = exbs[0]
                    else:
                        exrow = zerov
                        for h in range(H):
                            exrow = jnp.where(lane == h, exbs[h], exrow)
                        oddf = oddfv[j]
                        vvtf = vvtfv[j]
                        shifted = exrow + oddf * (_perm(exrow, 8) - exrow)
                        for vv in range(8):
                            m = jnp.maximum(
                                0.0, 1.0 - jnp.abs(vvtf - float(vv)))
                            wd[e, pl.ds(vv * 16, 16)] = shifted * m

        def _fire_idx(b, p):
            base = wid * EPW + b * KE
            pltpu.async_copy(eidx_hbm.at[pl.ds(base, KE)], S[p], SS[p])
            pltpu.async_copy(eidx_hbm.at[pl.ds(E + base, KD)], DD[p], SD[p])

        def _fire_gathers(p):
            pltpu.async_copy(xl_hbm.at[S[p]], GL[p], SGL[p])
            pltpu.async_copy(xr_hbm.at[DD[p]], GR[p], SGR[p])

        def _wait_idx(p):
            pltpu.make_async_copy(
                eidx_hbm.at[pl.ds(0, KE)], S[p], SS[p]).wait()
            pltpu.make_async_copy(
                eidx_hbm.at[pl.ds(0, KD)], DD[p], SD[p]).wait()

        def _wait_gathers(p):
            pltpu.make_async_copy(xl_hbm.at[S[p]], GL[p], SGL[p]).wait()
            pltpu.make_async_copy(xr_hbm.at[DD[p]], GR[p], SGR[p]).wait()

        def _do_block(b, p):
            q = 1 - p
            _wait_idx(q)
            _fire_gathers(q)
            _wait_gathers(p)
            if H > 1:
                for g in range(KD // 16):
                    dv = DD[p][pl.ds(g * 16, 16)]
                    didxv[pl.ds(g * 16, 16)] = (
                        lax.shift_right_logical(dv, 4) + NACC)
            _compute_block(GL[p], GR[p], DD[p])
            pltpu.sync_copy(wv, acc.at[DD[p]], add=True)
            if H > 1:
                pltpu.sync_copy(wd, acc.at[didxv], add=True)
            _fire_idx(b + 2, p)

        base0 = wid * EPW
        pltpu.sync_copy(eidx_hbm.at[pl.ds(base0, KE)], s0)
        pltpu.sync_copy(eidx_hbm.at[pl.ds(E + base0, KD)], d0)
        _fire_gathers(0)
        _fire_idx(1, 1)

        def _pair(k, carry):
            b0 = 2 * k
            _do_block(b0, 0)
            _do_block(b0 + 1, 1)
            return carry

        lax.fori_loop(0, NBLK // 2, _pair, 0)
        _wait_gathers(0)
        _wait_idx(1)
        plsc.subcore_barrier()

        for k in range(NCH):
            r0 = s * RPT + k * RCH
            pltpu.sync_copy(acc.at[pl.ds(r0, RCH)], rowbuf)
            pltpu.sync_copy(rowbuf, out_hbm.at[c, pl.ds(r0, RCH)])

    return edge_pass


_edge_pass_1 = _make_edge_pass(HEADS * HID, HEADS, 0)
_edge_pass_2 = _make_edge_pass(OUT_DIM, 1, OUT_DIM)

_RB = 400
_NG = N // _RB
_RBA = 512
_NGA = NACC // _RBA


def _tc_in_proj(x, Wl, Wr):
    def body(x_ref, wl_ref, wr_ref, xl_ref, xr_ref):
        xb = x_ref[...]
        xl_ref[...] = jnp.dot(xb, wl_ref[...], preferred_element_type=jnp.float32)
        xr_ref[...] = jnp.dot(xb, wr_ref[...], preferred_element_type=jnp.float32)

    return pl.pallas_call(
        body,
        grid=(_NG,),
        in_specs=[
            pl.BlockSpec((_RB, IN_DIM), lambda i: (i, 0)),
            pl.BlockSpec((IN_DIM, IN_DIM), lambda i: (0, 0)),
            pl.BlockSpec((IN_DIM, IN_DIM), lambda i: (0, 0)),
        ],
        out_specs=[
            pl.BlockSpec((_RB, IN_DIM), lambda i: (i, 0)),
            pl.BlockSpec((_RB, IN_DIM), lambda i: (i, 0)),
        ],
        out_shape=[
            jax.ShapeDtypeStruct((N, IN_DIM), jnp.float32),
            jax.ShapeDtypeStruct((N, IN_DIM), jnp.float32),
        ],
    )(x, Wl, Wr)


def _tc_mid(nd, den8, b1, W3l, W3r):
    D1 = HEADS * HID

    def body(num_ref, den_ref, b1_ref, wl_ref, wr_ref, o_ref):
        num = num_ref[0] + num_ref[1]
        den = den_ref[0] + den_ref[1]
        col = lax.broadcasted_iota(jnp.int32, (HEADS, D1), 1)
        row = lax.broadcasted_iota(jnp.int32, (HEADS, D1), 0)
        bmat = jnp.where(col // HID == row, 1.0, 0.0)
        den_b = jnp.dot(den, bmat, preferred_element_type=jnp.float32)
        h = num / (den_b + 1e-16) + b1_ref[...]
        h = jnp.where(h > 0.0, h, jnp.exp(h) - 1.0)
        o_ref[:, :OUT_DIM] = jnp.dot(
            h, wl_ref[...], preferred_element_type=jnp.float32)
        o_ref[:, OUT_DIM:] = jnp.dot(
            h, wr_ref[...], preferred_element_type=jnp.float32)

    return pl.pallas_call(
        body,
        grid=(_NGA,),
        in_specs=[
            pl.BlockSpec((NC, _RBA, 128), lambda i: (0, i, 0)),
            pl.BlockSpec((NC, _RBA, HEADS), lambda i: (0, i, 0)),
            pl.BlockSpec((1, D1), lambda i: (0, 0)),
            pl.BlockSpec((D1, OUT_DIM), lambda i: (0, 0)),
            pl.BlockSpec((D1, OUT_DIM), lambda i: (0, 0)),
        ],
        out_specs=pl.BlockSpec((_RBA, 2 * OUT_DIM), lambda i: (i, 0)),
        out_shape=jax.ShapeDtypeStruct((NACC, 2 * OUT_DIM), jnp.float32),
    )(nd, den8, b1, W3l, W3r)


def _tc_final(nd, b3):
    def body(nd_ref, b3_ref, o_ref, ls_ref):
        nd0 = nd_ref[0]
        ndb = nd_ref[1]
        num = nd0[:, :OUT_DIM] + ndb[:, :OUT_DIM]
        den = nd0[:, OUT_DIM:OUT_DIM + 1] + ndb[:, OUT_DIM:OUT_DIM + 1]
        o = num / (den + 1e-16) + b3_ref[...]
        o = jnp.where(o > 0.0, o, jnp.exp(o) - 1.0)
        m = jnp.max(o, axis=1, keepdims=True)
        lse = m + jnp.log(jnp.sum(jnp.exp(o - m), axis=1, keepdims=True))
        o_ref[...] = o
        ls_ref[...] = o - lse

    return pl.pallas_call(
        body,
        grid=(_NGA,),
        in_specs=[
            pl.BlockSpec((NC, _RBA, 128), lambda i: (0, i, 0)),
            pl.BlockSpec((1, OUT_DIM), lambda i: (0, 0)),
        ],
        out_specs=[
            pl.BlockSpec((_RBA, OUT_DIM), lambda i: (i, 0)),
            pl.BlockSpec((_RBA, OUT_DIM), lambda i: (i, 0)),
        ],
        out_shape=[
            jax.ShapeDtypeStruct((NACC, OUT_DIM), jnp.float32),
            jax.ShapeDtypeStruct((NACC, OUT_DIM), jnp.float32),
        ],
    )(nd, b3)


def kernel(x, edge_index, W1l, W1r, att1, b1, W3l, W3r, att3, b3):
    eidx = jnp.concatenate(
        [jnp.reshape(edge_index, (-1,)), jnp.zeros((128,), jnp.int32)])
    xl1, xr1 = _tc_in_proj(x, W1l, W1r)
    nd1 = _edge_pass_1(xl1, xr1, eidx, jnp.reshape(att1, (-1,)))
    den8 = jnp.reshape(nd1[:, NACC:, :], (NC, NACC, HEADS))
    hcat = _tc_mid(nd1, den8, jnp.reshape(b1, (1, -1)), W3l, W3r)
    nd2 = _edge_pass_2(hcat, hcat, eidx, jnp.reshape(att3, (-1,)))
    o, ls = _tc_final(nd2, jnp.reshape(b3, (1, -1)))
    return (o[:N], ls[:N])


# speedup vs baseline: 24.7389x; 1.0452x over previous
import functools

import jax
import jax.numpy as jnp
from jax import lax
from jax.experimental import pallas as pl
from jax.experimental.pallas import tpu as pltpu
from jax.experimental.pallas import tpu_sc as plsc

N = 10000
E = 320000
IN_DIM = 128
HID = 16
HEADS = 8
OUT_DIM = 64

NC = 2
NS = 16
NW = NC * NS
EPW = E // NW
KE = 40
KD = 48
NBLK = EPW // KE
NACC = 10240
DROWS = 640
RCH = 8


def _make_edge_pass(D, H, xroff):
    VPH = D // 16 // H
    NROWS = NACC + (DROWS if H > 1 else 0)
    RPT = NROWS // NS
    NCH = RPT // RCH

    mesh = plsc.VectorSubcoreMesh(core_axis_name="c", subcore_axis_name="s")

    scratch = [
        pltpu.VMEM((KE,), jnp.int32),
        pltpu.VMEM((KE,), jnp.int32),
        pltpu.VMEM((KD,), jnp.int32),
        pltpu.VMEM((KD,), jnp.int32),
        pltpu.VMEM((KE, 128), jnp.float32),
        pltpu.VMEM((KE, 128), jnp.float32),
        pltpu.VMEM((KD, 128), jnp.float32),
        pltpu.VMEM((KD, 128), jnp.float32),
        pltpu.VMEM(((2 * KD if H > 1 else KD), 128), jnp.float32),
        pltpu.VMEM((D,), jnp.float32),
        pltpu.VMEM((RCH, 128), jnp.float32),
        pltpu.VMEM_SHARED((NROWS, 128), jnp.float32),
        pltpu.SemaphoreType.DMA,
        pltpu.SemaphoreType.DMA,
        pltpu.SemaphoreType.DMA,
        pltpu.SemaphoreType.DMA,
        pltpu.SemaphoreType.DMA,
        pltpu.SemaphoreType.DMA,
        pltpu.SemaphoreType.DMA,
        pltpu.SemaphoreType.DMA,
    ]
    if H > 1:
        scratch += [
            pltpu.VMEM((2 * KD,), jnp.int32),
        ]

    @functools.partial(
        pl.kernel,
        mesh=mesh,
        out_type=jax.ShapeDtypeStruct((NC, NROWS, 128), jnp.float32),
        scratch_types=scratch,
    )
    def edge_pass(xl_hbm, xr_hbm, eidx_hbm, att_hbm, out_hbm,
                  s0, s1, d0, d1, gl0, gl1, gr0, gr1, wv, attv, rowbuf,
                  acc, ss0, ss1, sd0, sd1, sgl0, sgl1, sgr0, sgr1,
                  *den_scratch):
        if H > 1:
            (cidx,) = den_scratch
        S = (s0, s1)
        DD = (d0, d1)
        GL = (gl0, gl1)
        GR = (gr0, gr1)
        SS = (ss0, ss1)
        SD = (sd0, sd1)
        SGL = (sgl0, sgl1)
        SGR = (sgr0, sgr1)
        c = lax.axis_index("c")
        s = lax.axis_index("s")
        wid = c * NS + s

        pltpu.sync_copy(att_hbm, attv)

        zerov = jnp.zeros((16,), jnp.float32)

        def _zrow(i, carry):
            for j in range(8):
                rowbuf[i, pl.ds(j * 16, 16)] = zerov
            return carry
        lax.fori_loop(0, RCH, _zrow, 0)
        for k in range(NCH):
            r0 = s * RPT + k * RCH
            pltpu.sync_copy(rowbuf, acc.at[pl.ds(r0, RCH)])
        plsc.subcore_barrier()

        if H == 1:
            def _ztail(e, carry):
                for j in range(5, 8):
                    wv[e, pl.ds(j * 16, 16)] = zerov
                return carry
            lax.fori_loop(0, KD, _ztail, 0)
        tails = list(range(KE, KD))
        if H > 1:
            tails += list(range(KD + KE, 2 * KD))
        for e in tails:
            for j in range(8):
                wv[e, pl.ds(j * 16, 16)] = zerov

        attvs = [attv[pl.ds(v * 16, 16)] for v in range(D // 16)]
        lane = lax.iota(jnp.int32, 16)
        gdn = lax.GatherDimensionNumbers(
            offset_dims=(), collapsed_slice_dims=(0,), start_index_map=(0,))

        def _perm(u, xor):
            return lax.gather(u, (lane ^ xor)[:, None], gdn, (1,),
                              mode=lax.GatherScatterMode.PROMISE_IN_BOUNDS)

        def _hsum(u):
            for k2 in (8, 4, 2, 1):
                u = u + _perm(u, k2)
            return u

        def _compute_block(xlr, xrr, dstv):
            def _group(g, jmax):
                if H > 1:
                    dgrp = dstv[pl.ds(g * 16, 16)]
                    og = dgrp & 15
                    oddfv = (og & 1).astype(jnp.float32)
                    vvtfv = lax.shift_right_logical(og, 1).astype(
                        jnp.float32)
                for j in range(jmax):
                    e = g * 16 + j
                    exbs = []
                    for h in range(H):
                        usum = None
                        xls = []
                        for jj in range(VPH):
                            v = h * VPH + jj
                            xlv = xlr[e, pl.ds(v * 16, 16)]
                            xrv = xrr[e, pl.ds(xroff + v * 16, 16)]
                            t = xlv + xrv
                            t = jnp.where(t >= 0.0, t, t * 0.2)
                            u = t * attvs[v]
                            usum = u if usum is None else usum + u
                            xls.append(xlv)
                        exb = jnp.exp(_hsum(usum))
                        exbs.append(exb)
                        for jj in range(VPH):
                            v = h * VPH + jj
                            wv[e, pl.ds(v * 16, 16)] = xls[jj] * exb
                    if H == 1:
                        wv[e, pl.ds(D, 16)] = exbs[0]
                    else:
                        exrow = zerov
                        for h in range(H):
                            exrow = jnp.where(lane == h, exbs[h], exrow)
                        oddf = oddfv[j]
                        vvtf = vvtfv[j]
                        shifted = exrow + oddf * (_perm(exrow, 8) - exrow)
                        for vv in range(8):
                            m = jnp.maximum(
                                0.0, 1.0 - jnp.abs(vvtf - float(vv)))
                            wv[KD + e, pl.ds(vv * 16, 16)] = shifted * m

            def _gbody(g, carry):
                _group(g, 16)
                return carry
            lax.fori_loop(0, 2, _gbody, 0)
            _group(2, 8)

        def _fire_idx(b, p):
            base = wid * EPW + b * KE
            pltpu.async_copy(eidx_hbm.at[pl.ds(base, KE)], S[p], SS[p])
            pltpu.async_copy(eidx_hbm.at[pl.ds(E + base, KD)], DD[p], SD[p])

        def _fire_gathers(p):
            pltpu.async_copy(xl_hbm.at[S[p]], GL[p], SGL[p])
            pltpu.async_copy(xr_hbm.at[DD[p]], GR[p], SGR[p])

        def _wait_idx(p):
            pltpu.make_async_copy(
                eidx_hbm.at[pl.ds(0, KE)], S[p], SS[p]).wait()
            pltpu.make_async_copy(
                eidx_hbm.at[pl.ds(0, KD)], DD[p], SD[p]).wait()

        def _wait_gathers(p):
            pltpu.make_async_copy(xl_hbm.at[S[p]], GL[p], SGL[p]).wait()
            pltpu.make_async_copy(xr_hbm.at[DD[p]], GR[p], SGR[p]).wait()

        def _do_block(b, p):
            q = 1 - p
            _wait_idx(q)
            _fire_gathers(q)
            _wait_gathers(p)
            _compute_block(GL[p], GR[p], DD[p])
            if H > 1:
                for g in range(KD // 16):
                    dv = DD[p][pl.ds(g * 16, 16)]
                    cidx[pl.ds(g * 16, 16)] = dv
                    cidx[pl.ds(KD + g * 16, 16)] = (
                        lax.shift_right_logical(dv, 4) + NACC)
                pltpu.sync_copy(wv, acc.at[cidx], add=True)
            else:
                pltpu.sync_copy(wv, acc.at[DD[p]], add=True)
            _fire_idx(b + 2, p)

        base0 = wid * EPW
        pltpu.sync_copy(eidx_hbm.at[pl.ds(base0, KE)], s0)
        pltpu.sync_copy(eidx_hbm.at[pl.ds(E + base0, KD)], d0)
        _fire_gathers(0)
        _fire_idx(1, 1)

        def _pair(k, carry):
            b0 = 2 * k
            _do_block(b0, 0)
            _do_block(b0 + 1, 1)
            return carry

        lax.fori_loop(0, NBLK // 2, _pair, 0)
        _wait_gathers(0)
        _wait_idx(1)
        plsc.subcore_barrier()

        for k in range(NCH):
            r0 = s * RPT + k * RCH
            pltpu.sync_copy(acc.at[pl.ds(r0, RCH)], rowbuf)
            pltpu.sync_copy(rowbuf, out_hbm.at[c, pl.ds(r0, RCH)])

    return edge_pass


_edge_pass_1 = _make_edge_pass(HEADS * HID, HEADS, 0)
_edge_pass_2 = _make_edge_pass(OUT_DIM, 1, OUT_DIM)

_RB = 400
_NG = N // _RB
_RBA = 512
_NGA = NACC // _RBA


def _tc_in_proj(x, Wl, Wr):
    def body(x_ref, wl_ref, wr_ref, xl_ref, xr_ref):
        xb = x_ref[...]
        xl_ref[...] = jnp.dot(xb, wl_ref[...], preferred_element_type=jnp.float32)
        xr_ref[...] = jnp.dot(xb, wr_ref[...], preferred_element_type=jnp.float32)

    return pl.pallas_call(
        body,
        grid=(_NG,),
        in_specs=[
            pl.BlockSpec((_RB, IN_DIM), lambda i: (i, 0)),
            pl.BlockSpec((IN_DIM, IN_DIM), lambda i: (0, 0)),
            pl.BlockSpec((IN_DIM, IN_DIM), lambda i: (0, 0)),
        ],
        out_specs=[
            pl.BlockSpec((_RB, IN_DIM), lambda i: (i, 0)),
            pl.BlockSpec((_RB, IN_DIM), lambda i: (i, 0)),
        ],
        out_shape=[
            jax.ShapeDtypeStruct((N, IN_DIM), jnp.float32),
            jax.ShapeDtypeStruct((N, IN_DIM), jnp.float32),
        ],
    )(x, Wl, Wr)


def _tc_mid(nd, den8, b1, W3l, W3r):
    D1 = HEADS * HID

    def body(num_ref, den_ref, b1_ref, wl_ref, wr_ref, o_ref):
        num = num_ref[0] + num_ref[1]
        den = den_ref[0] + den_ref[1]
        col = lax.broadcasted_iota(jnp.int32, (HEADS, D1), 1)
        row = lax.broadcasted_iota(jnp.int32, (HEADS, D1), 0)
        bmat = jnp.where(col // HID == row, 1.0, 0.0)
        den_b = jnp.dot(den, bmat, preferred_element_type=jnp.float32)
        h = num / (den_b + 1e-16) + b1_ref[...]
        h = jnp.where(h > 0.0, h, jnp.exp(h) - 1.0)
        o_ref[:, :OUT_DIM] = jnp.dot(
            h, wl_ref[...], preferred_element_type=jnp.float32)
        o_ref[:, OUT_DIM:] = jnp.dot(
            h, wr_ref[...], preferred_element_type=jnp.float32)

    return pl.pallas_call(
        body,
        grid=(_NGA,),
        in_specs=[
            pl.BlockSpec((NC, _RBA, 128), lambda i: (0, i, 0)),
            pl.BlockSpec((NC, _RBA, HEADS), lambda i: (0, i, 0)),
            pl.BlockSpec((1, D1), lambda i: (0, 0)),
            pl.BlockSpec((D1, OUT_DIM), lambda i: (0, 0)),
            pl.BlockSpec((D1, OUT_DIM), lambda i: (0, 0)),
        ],
        out_specs=pl.BlockSpec((_RBA, 2 * OUT_DIM), lambda i: (i, 0)),
        out_shape=jax.ShapeDtypeStruct((NACC, 2 * OUT_DIM), jnp.float32),
    )(nd, den8, b1, W3l, W3r)


def _tc_final(nd, b3):
    def body(nd_ref, b3_ref, o_ref, ls_ref):
        nd0 = nd_ref[0]
        ndb = nd_ref[1]
        num = nd0[:, :OUT_DIM] + ndb[:, :OUT_DIM]
        den = nd0[:, OUT_DIM:OUT_DIM + 1] + ndb[:, OUT_DIM:OUT_DIM + 1]
        o = num / (den + 1e-16) + b3_ref[...]
        o = jnp.where(o > 0.0, o, jnp.exp(o) - 1.0)
        m = jnp.max(o, axis=1, keepdims=True)
        lse = m + jnp.log(jnp.sum(jnp.exp(o - m), axis=1, keepdims=True))
        o_ref[...] = o
        ls_ref[...] = o - lse

    return pl.pallas_call(
        body,
        grid=(_NGA,),
        in_specs=[
            pl.BlockSpec((NC, _RBA, 128), lambda i: (0, i, 0)),
            pl.BlockSpec((1, OUT_DIM), lambda i: (0, 0)),
        ],
        out_specs=[
            pl.BlockSpec((_RBA, OUT_DIM), lambda i: (i, 0)),
            pl.BlockSpec((_RBA, OUT_DIM), lambda i: (i, 0)),
        ],
        out_shape=[
            jax.ShapeDtypeStruct((NACC, OUT_DIM), jnp.float32),
            jax.ShapeDtypeStruct((NACC, OUT_DIM), jnp.float32),
        ],
    )(nd, b3)


def kernel(x, edge_index, W1l, W1r, att1, b1, W3l, W3r, att3, b3):
    eidx = jnp.concatenate(
        [jnp.reshape(edge_index, (-1,)), jnp.zeros((128,), jnp.int32)])
    xl1, xr1 = _tc_in_proj(x, W1l, W1r)
    nd1 = _edge_pass_1(xl1, xr1, eidx, jnp.reshape(att1, (-1,)))
    den8 = jnp.reshape(nd1[:, NACC:, :], (NC, NACC, HEADS))
    hcat = _tc_mid(nd1, den8, jnp.reshape(b1, (1, -1)), W3l, W3r)
    nd2 = _edge_pass_2(hcat, hcat, eidx, jnp.reshape(att3, (-1,)))
    o, ls = _tc_final(nd2, jnp.reshape(b3, (1, -1)))
    return (o[:N], ls[:N])


# speedup vs baseline: 25.2271x; 1.0197x over previous
import functools

import jax
import jax.numpy as jnp
from jax import lax
from jax.experimental import pallas as pl
from jax.experimental.pallas import tpu as pltpu
from jax.experimental.pallas import tpu_sc as plsc

N = 10000
E = 320000
IN_DIM = 128
HID = 16
HEADS = 8
OUT_DIM = 64

NC = 2
NS = 16
NW = NC * NS
EPW = E // NW
KE = 40
KD = 48
NBLK = EPW // KE
NACC = 10240
DROWS = 640
RCH = 8


def _make_edge_pass(D, H, xroff):
    VPH = D // 16 // H
    NROWS = NACC + (DROWS if H > 1 else 0)
    RPT = NROWS // NS
    NCH = RPT // RCH

    mesh = plsc.VectorSubcoreMesh(core_axis_name="c", subcore_axis_name="s")

    scratch = [
        pltpu.VMEM((KE,), jnp.int32),
        pltpu.VMEM((KE,), jnp.int32),
        pltpu.VMEM((KD,), jnp.int32),
        pltpu.VMEM((KD,), jnp.int32),
        pltpu.VMEM((KE, 128), jnp.float32),
        pltpu.VMEM((KE, 128), jnp.float32),
        pltpu.VMEM((KD, 128), jnp.float32),
        pltpu.VMEM((KD, 128), jnp.float32),
        pltpu.VMEM(((2 * KD if H > 1 else KD), 128), jnp.float32),
        pltpu.VMEM((D,), jnp.float32),
        pltpu.VMEM((RCH, 128), jnp.float32),
        pltpu.VMEM_SHARED((NROWS, 128), jnp.float32),
        pltpu.SemaphoreType.DMA,
        pltpu.SemaphoreType.DMA,
        pltpu.SemaphoreType.DMA,
        pltpu.SemaphoreType.DMA,
        pltpu.SemaphoreType.DMA,
        pltpu.SemaphoreType.DMA,
        pltpu.SemaphoreType.DMA,
        pltpu.SemaphoreType.DMA,
    ]
    if H > 1:
        scratch += [
            pltpu.VMEM((2 * KD,), jnp.int32),
        ]

    @functools.partial(
        pl.kernel,
        mesh=mesh,
        out_type=jax.ShapeDtypeStruct((NC, NROWS, 128), jnp.float32),
        scratch_types=scratch,
    )
    def edge_pass(xl_hbm, xr_hbm, eidx_hbm, att_hbm, out_hbm,
                  s0, s1, d0, d1, gl0, gl1, gr0, gr1, wv, attv, rowbuf,
                  acc, ss0, ss1, sd0, sd1, sgl0, sgl1, sgr0, sgr1,
                  *den_scratch):
        if H > 1:
            (cidx,) = den_scratch
        S = (s0, s1)
        DD = (d0, d1)
        GL = (gl0, gl1)
        GR = (gr0, gr1)
        SS = (ss0, ss1)
        SD = (sd0, sd1)
        SGL = (sgl0, sgl1)
        SGR = (sgr0, sgr1)
        c = lax.axis_index("c")
        s = lax.axis_index("s")
        wid = c * NS + s

        pltpu.sync_copy(att_hbm, attv)

        zerov = jnp.zeros((16,), jnp.float32)

        def _zrow(i, carry):
            for j in range(8):
                rowbuf[i, pl.ds(j * 16, 16)] = zerov
            return carry
        lax.fori_loop(0, RCH, _zrow, 0)

        def _zfire(k, carry):
            r0 = s * RPT + k * RCH
            pltpu.async_copy(rowbuf, acc.at[pl.ds(r0, RCH)], ss0)
            return carry

        def _zdrain(k, carry):
            r0 = s * RPT + k * RCH
            pltpu.make_async_copy(
                rowbuf, acc.at[pl.ds(r0, RCH)], ss0).wait()
            return carry

        lax.fori_loop(0, NCH, _zfire, 0)
        lax.fori_loop(0, NCH, _zdrain, 0)
        plsc.subcore_barrier()

        if H == 1:
            def _ztail(e, carry):
                for j in range(5, 8):
                    wv[e, pl.ds(j * 16, 16)] = zerov
                return carry
            lax.fori_loop(0, KD, _ztail, 0)
        tails = list(range(KE, KD))
        if H > 1:
            tails += list(range(KD + KE, 2 * KD))
        for e in tails:
            for j in range(8):
                wv[e, pl.ds(j * 16, 16)] = zerov

        attvs = [attv[pl.ds(v * 16, 16)] for v in range(D // 16)]
        lane = lax.iota(jnp.int32, 16)
        gdn = lax.GatherDimensionNumbers(
            offset_dims=(), collapsed_slice_dims=(0,), start_index_map=(0,))

        def _perm(u, xor):
            return lax.gather(u, (lane ^ xor)[:, None], gdn, (1,),
                              mode=lax.GatherScatterMode.PROMISE_IN_BOUNDS)

        def _hsum(u):
            for k2 in (8, 4, 2, 1):
                u = u + _perm(u, k2)
            return u

        def _compute_block(xlr, xrr, dstv):
            def _group(g, jmax):
                if H > 1:
                    dgrp = dstv[pl.ds(g * 16, 16)]
                    og = dgrp & 15
                    oddfv = (og & 1).astype(jnp.float32)
                    vvtfv = lax.shift_right_logical(og, 1).astype(
                        jnp.float32)
                for j in range(jmax):
                    e = g * 16 + j
                    exbs = []
                    for h in range(H):
                        usum = None
                        xls = []
                        for jj in range(VPH):
                            v = h * VPH + jj
                            xlv = xlr[e, pl.ds(v * 16, 16)]
                            xrv = xrr[e, pl.ds(xroff + v * 16, 16)]
                            t = xlv + xrv
                            t = jnp.where(t >= 0.0, t, t * 0.2)
                            u = t * attvs[v]
                            usum = u if usum is None else usum + u
                            xls.append(xlv)
                        exb = jnp.exp(_hsum(usum))
                        exbs.append(exb)
                        for jj in range(VPH):
                            v = h * VPH + jj
                            wv[e, pl.ds(v * 16, 16)] = xls[jj] * exb
                    if H == 1:
                        wv[e, pl.ds(D, 16)] = exbs[0]
                    else:
                        exrow = zerov
                        for h in range(H):
                            exrow = jnp.where(lane == h, exbs[h], exrow)
                        oddf = oddfv[j]
                        vvtf = vvtfv[j]
                        shifted = exrow + oddf * (_perm(exrow, 8) - exrow)
                        for vv in range(8):
                            m = jnp.maximum(
                                0.0, 1.0 - jnp.abs(vvtf - float(vv)))
                            wv[KD + e, pl.ds(vv * 16, 16)] = shifted * m

            def _gbody(g, carry):
                _group(g, 16)
                return carry
            lax.fori_loop(0, 2, _gbody, 0)
            _group(2, 8)

        def _fire_idx(b, p):
            base = wid * EPW + b * KE
            pltpu.async_copy(eidx_hbm.at[pl.ds(base, KE)], S[p], SS[p])
            pltpu.async_copy(eidx_hbm.at[pl.ds(E + base, KD)], DD[p], SD[p])

        def _fire_gathers(p):
            pltpu.async_copy(xl_hbm.at[S[p]], GL[p], SGL[p])
            pltpu.async_copy(xr_hbm.at[DD[p]], GR[p], SGR[p])

        def _wait_idx(p):
            pltpu.make_async_copy(
                eidx_hbm.at[pl.ds(0, KE)], S[p], SS[p]).wait()
            pltpu.make_async_copy(
                eidx_hbm.at[pl.ds(0, KD)], DD[p], SD[p]).wait()

        def _wait_gathers(p):
            pltpu.make_async_copy(xl_hbm.at[S[p]], GL[p], SGL[p]).wait()
            pltpu.make_async_copy(xr_hbm.at[DD[p]], GR[p], SGR[p]).wait()

        def _do_block(b, p):
            q = 1 - p
            _wait_idx(q)
            _fire_gathers(q)
            _wait_gathers(p)
            _compute_block(GL[p], GR[p], DD[p])
            if H > 1:
                for g in range(KD // 16):
                    dv = DD[p][pl.ds(g * 16, 16)]
                    cidx[pl.ds(g * 16, 16)] = dv
                    cidx[pl.ds(KD + g * 16, 16)] = (
                        lax.shift_right_logical(dv, 4) + NACC)
                pltpu.sync_copy(wv, acc.at[cidx], add=True)
            else:
                pltpu.sync_copy(wv, acc.at[DD[p]], add=True)
            _fire_idx(b + 2, p)

        base0 = wid * EPW
        pltpu.sync_copy(eidx_hbm.at[pl.ds(base0, KE)], s0)
        pltpu.sync_copy(eidx_hbm.at[pl.ds(E + base0, KD)], d0)
        _fire_gathers(0)
        _fire_idx(1, 1)

        def _pair(k, carry):
            b0 = 2 * k
            _do_block(b0, 0)
            _do_block(b0 + 1, 1)
            return carry

        lax.fori_loop(0, NBLK // 2, _pair, 0)
        _wait_gathers(0)
        _wait_idx(1)
        plsc.subcore_barrier()

        r0 = s * RPT
        pltpu.sync_copy(acc.at[pl.ds(r0, RPT)],
                        out_hbm.at[c, pl.ds(r0, RPT)])

    return edge_pass


_edge_pass_1 = _make_edge_pass(HEADS * HID, HEADS, 0)
_edge_pass_2 = _make_edge_pass(OUT_DIM, 1, OUT_DIM)

_RB = 400
_NG = N // _RB
_RBA = 512
_NGA = NACC // _RBA


def _tc_in_proj(x, Wl, Wr):
    def body(x_ref, wl_ref, wr_ref, xl_ref, xr_ref):
        xb = x_ref[...]
        xl_ref[...] = jnp.dot(xb, wl_ref[...], preferred_element_type=jnp.float32)
        xr_ref[...] = jnp.dot(xb, wr_ref[...], preferred_element_type=jnp.float32)

    return pl.pallas_call(
        body,
        grid=(_NG,),
        in_specs=[
            pl.BlockSpec((_RB, IN_DIM), lambda i: (i, 0)),
            pl.BlockSpec((IN_DIM, IN_DIM), lambda i: (0, 0)),
            pl.BlockSpec((IN_DIM, IN_DIM), lambda i: (0, 0)),
        ],
        out_specs=[
            pl.BlockSpec((_RB, IN_DIM), lambda i: (i, 0)),
            pl.BlockSpec((_RB, IN_DIM), lambda i: (i, 0)),
        ],
        out_shape=[
            jax.ShapeDtypeStruct((N, IN_DIM), jnp.float32),
            jax.ShapeDtypeStruct((N, IN_DIM), jnp.float32),
        ],
    )(x, Wl, Wr)


def _tc_mid(nd, den8, b1, W3l, W3r):
    D1 = HEADS * HID

    def body(num_ref, den_ref, b1_ref, wl_ref, wr_ref, o_ref):
        num = num_ref[0] + num_ref[1]
        den = den_ref[0] + den_ref[1]
        col = lax.broadcasted_iota(jnp.int32, (HEADS, D1), 1)
        row = lax.broadcasted_iota(jnp.int32, (HEADS, D1), 0)
        bmat = jnp.where(col // HID == row, 1.0, 0.0)
        den_b = jnp.dot(den, bmat, preferred_element_type=jnp.float32)
        h = num / (den_b + 1e-16) + b1_ref[...]
        h = jnp.where(h > 0.0, h, jnp.exp(h) - 1.0)
        o_ref[:, :OUT_DIM] = jnp.dot(
            h, wl_ref[...], preferred_element_type=jnp.float32)
        o_ref[:, OUT_DIM:] = jnp.dot(
            h, wr_ref[...], preferred_element_type=jnp.float32)

    return pl.pallas_call(
        body,
        grid=(_NGA,),
        in_specs=[
            pl.BlockSpec((NC, _RBA, 128), lambda i: (0, i, 0)),
            pl.BlockSpec((NC, _RBA, HEADS), lambda i: (0, i, 0)),
            pl.BlockSpec((1, D1), lambda i: (0, 0)),
            pl.BlockSpec((D1, OUT_DIM), lambda i: (0, 0)),
            pl.BlockSpec((D1, OUT_DIM), lambda i: (0, 0)),
        ],
        out_specs=pl.BlockSpec((_RBA, 2 * OUT_DIM), lambda i: (i, 0)),
        out_shape=jax.ShapeDtypeStruct((NACC, 2 * OUT_DIM), jnp.float32),
    )(nd, den8, b1, W3l, W3r)


def _tc_final(nd, b3):
    def body(nd_ref, b3_ref, o_ref, ls_ref):
        nd0 = nd_ref[0]
        ndb = nd_ref[1]
        num = nd0[:, :OUT_DIM] + ndb[:, :OUT_DIM]
        den = nd0[:, OUT_DIM:OUT_DIM + 1] + ndb[:, OUT_DIM:OUT_DIM + 1]
        o = num / (den + 1e-16) + b3_ref[...]
        o = jnp.where(o > 0.0, o, jnp.exp(o) - 1.0)
        m = jnp.max(o, axis=1, keepdims=True)
        lse = m + jnp.log(jnp.sum(jnp.exp(o - m), axis=1, keepdims=True))
        o_ref[...] = o
        ls_ref[...] = o - lse

    return pl.pallas_call(
        body,
        grid=(_NGA,),
        in_specs=[
            pl.BlockSpec((NC, _RBA, 128), lambda i: (0, i, 0)),
            pl.BlockSpec((1, OUT_DIM), lambda i: (0, 0)),
        ],
        out_specs=[
            pl.BlockSpec((_RBA, OUT_DIM), lambda i: (i, 0)),
            pl.BlockSpec((_RBA, OUT_DIM), lambda i: (i, 0)),
        ],
        out_shape=[
            jax.ShapeDtypeStruct((NACC, OUT_DIM), jnp.float32),
            jax.ShapeDtypeStruct((NACC, OUT_DIM), jnp.float32),
        ],
    )(nd, b3)


def kernel(x, edge_index, W1l, W1r, att1, b1, W3l, W3r, att3, b3):
    eidx = jnp.concatenate(
        [jnp.reshape(edge_index, (-1,)), jnp.zeros((128,), jnp.int32)])
    xl1, xr1 = _tc_in_proj(x, W1l, W1r)
    nd1 = _edge_pass_1(xl1, xr1, eidx, jnp.reshape(att1, (-1,)))
    den8 = jnp.reshape(nd1[:, NACC:, :], (NC, NACC, HEADS))
    hcat = _tc_mid(nd1, den8, jnp.reshape(b1, (1, -1)), W3l, W3r)
    nd2 = _edge_pass_2(hcat, hcat, eidx, jnp.reshape(att3, (-1,)))
    o, ls = _tc_final(nd2, jnp.reshape(b3, (1, -1)))
    return (o[:N], ls[:N])


# speedup vs baseline: 26.1380x; 1.0361x over previous
import functools

import jax
import jax.numpy as jnp
from jax import lax
from jax.experimental import pallas as pl
from jax.experimental.pallas import tpu as pltpu
from jax.experimental.pallas import tpu_sc as plsc

N = 10000
E = 320000
IN_DIM = 128
HID = 16
HEADS = 8
OUT_DIM = 64

NC = 2
NS = 16
NW = NC * NS
EPW = E // NW
KE = 40
KD = 48
NBLK = EPW // KE
NACC = 10240
DROWS = 640
RCH = 8


def _make_edge_pass(D, H, xroff):
    VPH = D // 16 // H
    NROWS = NACC + (DROWS if H > 1 else 0)
    RPT = NROWS // NS
    NCH = RPT // RCH

    mesh = plsc.VectorSubcoreMesh(core_axis_name="c", subcore_axis_name="s")

    scratch = [
        pltpu.VMEM((KE,), jnp.int32),
        pltpu.VMEM((KE,), jnp.int32),
        pltpu.VMEM((KD,), jnp.int32),
        pltpu.VMEM((KD,), jnp.int32),
        pltpu.VMEM((KE, 128), jnp.float32),
        pltpu.VMEM((KE, 128), jnp.float32),
        pltpu.VMEM((KD, 128), jnp.float32),
        pltpu.VMEM((KD, 128), jnp.float32),
        pltpu.VMEM(((2 * KD if H > 1 else KD), 128), jnp.float32),
        pltpu.VMEM((D,), jnp.float32),
        pltpu.VMEM((RCH, 128), jnp.float32),
        pltpu.VMEM_SHARED((NROWS, 128), jnp.float32),
        pltpu.SemaphoreType.DMA,
        pltpu.SemaphoreType.DMA,
        pltpu.SemaphoreType.DMA,
        pltpu.SemaphoreType.DMA,
        pltpu.SemaphoreType.DMA,
        pltpu.SemaphoreType.DMA,
        pltpu.SemaphoreType.DMA,
        pltpu.SemaphoreType.DMA,
    ]
    if H > 1:
        scratch += [
            pltpu.VMEM((2 * KD,), jnp.int32),
        ]
    else:
        scratch += [
            pltpu.VMEM((KD, 128), jnp.float32),
            pltpu.SemaphoreType.DMA,
            pltpu.SemaphoreType.DMA,
        ]

    @functools.partial(
        pl.kernel,
        mesh=mesh,
        out_type=jax.ShapeDtypeStruct((NC, NROWS, 128), jnp.float32),
        scratch_types=scratch,
    )
    def edge_pass(xl_hbm, xr_hbm, eidx_hbm, att_hbm, out_hbm,
                  s0, s1, d0, d1, gl0, gl1, gr0, gr1, wv, attv, rowbuf,
                  acc, ss0, ss1, sd0, sd1, sgl0, sgl1, sgr0, sgr1,
                  *den_scratch):
        if H > 1:
            (cidx,) = den_scratch
            wvr = (wv, wv)
        else:
            wv2, ssc0, ssc1 = den_scratch
            wvr = (wv, wv2)
            SSC = (ssc0, ssc1)
        S = (s0, s1)
        DD = (d0, d1)
        GL = (gl0, gl1)
        GR = (gr0, gr1)
        SS = (ss0, ss1)
        SD = (sd0, sd1)
        SGL = (sgl0, sgl1)
        SGR = (sgr0, sgr1)
        c = lax.axis_index("c")
        s = lax.axis_index("s")
        wid = c * NS + s

        pltpu.sync_copy(att_hbm, attv)

        zerov = jnp.zeros((16,), jnp.float32)

        def _zrow(i, carry):
            for j in range(8):
                rowbuf[i, pl.ds(j * 16, 16)] = zerov
            return carry
        lax.fori_loop(0, RCH, _zrow, 0)

        def _zfire(k, carry):
            r0 = s * RPT + k * RCH
            pltpu.async_copy(rowbuf, acc.at[pl.ds(r0, RCH)], ss0)
            return carry

        def _zdrain(k, carry):
            r0 = s * RPT + k * RCH
            pltpu.make_async_copy(
                rowbuf, acc.at[pl.ds(r0, RCH)], ss0).wait()
            return carry

        lax.fori_loop(0, NCH, _zfire, 0)
        lax.fori_loop(0, NCH, _zdrain, 0)
        plsc.subcore_barrier()

        if H == 1:
            def _ztail(e, carry):
                for j in range(5, 8):
                    wv[e, pl.ds(j * 16, 16)] = zerov
                    wv2[e, pl.ds(j * 16, 16)] = zerov
                return carry
            lax.fori_loop(0, KD, _ztail, 0)
        tails = list(range(KE, KD))
        if H > 1:
            tails += list(range(KD + KE, 2 * KD))
        for e in tails:
            for j in range(8):
                wv[e, pl.ds(j * 16, 16)] = zerov
                if H == 1:
                    wv2[e, pl.ds(j * 16, 16)] = zerov

        attvs = [attv[pl.ds(v * 16, 16)] for v in range(D // 16)]
        lane = lax.iota(jnp.int32, 16)
        gdn = lax.GatherDimensionNumbers(
            offset_dims=(), collapsed_slice_dims=(0,), start_index_map=(0,))

        def _perm(u, xor):
            return lax.gather(u, (lane ^ xor)[:, None], gdn, (1,),
                              mode=lax.GatherScatterMode.PROMISE_IN_BOUNDS)

        def _hsum(u):
            for k2 in (8, 4, 2, 1):
                u = u + _perm(u, k2)
            return u

        def _compute_block(xlr, xrr, dstv, wvt):
            def _group(g, jmax):
                if H > 1:
                    dgrp = dstv[pl.ds(g * 16, 16)]
                    og = dgrp & 15
                    oddfv = (og & 1).astype(jnp.float32)
                    vvtfv = lax.shift_right_logical(og, 1).astype(
                        jnp.float32)
                for j in range(jmax):
                    e = g * 16 + j
                    exbs = []
                    for h in range(H):
                        usum = None
                        xls = []
                        for jj in range(VPH):
                            v = h * VPH + jj
                            xlv = xlr[e, pl.ds(v * 16, 16)]
                            xrv = xrr[e, pl.ds(xroff + v * 16, 16)]
                            t = xlv + xrv
                            t = jnp.where(t >= 0.0, t, t * 0.2)
                            u = t * attvs[v]
                            usum = u if usum is None else usum + u
                            xls.append(xlv)
                        exb = jnp.exp(_hsum(usum))
                        exbs.append(exb)
                        for jj in range(VPH):
                            v = h * VPH + jj
                            wvt[e, pl.ds(v * 16, 16)] = xls[jj] * exb
                    if H == 1:
                        wvt[e, pl.ds(D, 16)] = exbs[0]
                    else:
                        exrow = zerov
                        for h in range(H):
                            exrow = jnp.where(lane == h, exbs[h], exrow)
                        oddf = oddfv[j]
                        vvtf = vvtfv[j]
                        shifted = exrow + oddf * (_perm(exrow, 8) - exrow)
                        for vv in range(8):
                            m = jnp.maximum(
                                0.0, 1.0 - jnp.abs(vvtf - float(vv)))
                            wvt[KD + e, pl.ds(vv * 16, 16)] = shifted * m

            def _gbody(g, carry):
                _group(g, 16)
                return carry
            lax.fori_loop(0, 2, _gbody, 0)
            _group(2, 8)

        def _fire_idx(b, p):
            base = wid * EPW + b * KE
            pltpu.async_copy(eidx_hbm.at[pl.ds(base, KE)], S[p], SS[p])
            pltpu.async_copy(eidx_hbm.at[pl.ds(E + base, KD)], DD[p], SD[p])

        def _fire_gathers(p):
            pltpu.async_copy(xl_hbm.at[S[p]], GL[p], SGL[p])
            pltpu.async_copy(xr_hbm.at[DD[p]], GR[p], SGR[p])

        def _wait_idx(p):
            pltpu.make_async_copy(
                eidx_hbm.at[pl.ds(0, KE)], S[p], SS[p]).wait()
            pltpu.make_async_copy(
                eidx_hbm.at[pl.ds(0, KD)], DD[p], SD[p]).wait()

        def _wait_gathers(p):
            pltpu.make_async_copy(xl_hbm.at[S[p]], GL[p], SGL[p]).wait()
            pltpu.make_async_copy(xr_hbm.at[DD[p]], GR[p], SGR[p]).wait()

        def _do_block(b, p):
            q = 1 - p
            _wait_idx(q)
            _fire_gathers(q)
            _wait_gathers(p)
            _compute_block(GL[p], GR[p], DD[p], wvr[p])
            if H > 1:
                for g in range(KD // 16):
                    dv = DD[p][pl.ds(g * 16, 16)]
                    cidx[pl.ds(g * 16, 16)] = dv
                    cidx[pl.ds(KD + g * 16, 16)] = (
                        lax.shift_right_logical(dv, 4) + NACC)
                pltpu.sync_copy(wv, acc.at[cidx], add=True)
            else:
                pltpu.async_copy(wvr[p], acc.at[DD[p]], SSC[p], add=True)
            _fire_idx(b + 2, p)

        def _wait_scat(p):
            pltpu.make_async_copy(wvr[p], acc.at[DD[p]], SSC[p]).wait()

        base0 = wid * EPW
        pltpu.sync_copy(eidx_hbm.at[pl.ds(base0, KE)], s0)
        pltpu.sync_copy(eidx_hbm.at[pl.ds(E + base0, KD)], d0)
        _fire_gathers(0)
        _fire_idx(1, 1)

        def _pair(k, carry):
            if H == 1:
                @pl.when(k > 0)
                def _():
                    _wait_scat(0)
                    _wait_scat(1)
            b0 = 2 * k
            _do_block(b0, 0)
            _do_block(b0 + 1, 1)
            return carry

        lax.fori_loop(0, NBLK // 2, _pair, 0)
        _wait_gathers(0)
        _wait_idx(1)
        if H == 1:
            _wait_scat(0)
            _wait_scat(1)
        plsc.subcore_barrier()

        r0 = s * RPT
        pltpu.sync_copy(acc.at[pl.ds(r0, RPT)],
                        out_hbm.at[c, pl.ds(r0, RPT)])

    return edge_pass


_edge_pass_1 = _make_edge_pass(HEADS * HID, HEADS, 0)
_edge_pass_2 = _make_edge_pass(OUT_DIM, 1, OUT_DIM)

_RB = 400
_NG = N // _RB
_RBA = 512
_NGA = NACC // _RBA


def _tc_in_proj(x, Wl, Wr):
    def body(x_ref, wl_ref, wr_ref, xl_ref, xr_ref):
        xb = x_ref[...]
        xl_ref[...] = jnp.dot(xb, wl_ref[...], preferred_element_type=jnp.float32)
        xr_ref[...] = jnp.dot(xb, wr_ref[...], preferred_element_type=jnp.float32)

    return pl.pallas_call(
        body,
        grid=(_NG,),
        in_specs=[
            pl.BlockSpec((_RB, IN_DIM), lambda i: (i, 0)),
            pl.BlockSpec((IN_DIM, IN_DIM), lambda i: (0, 0)),
            pl.BlockSpec((IN_DIM, IN_DIM), lambda i: (0, 0)),
        ],
        out_specs=[
            pl.BlockSpec((_RB, IN_DIM), lambda i: (i, 0)),
            pl.BlockSpec((_RB, IN_DIM), lambda i: (i, 0)),
        ],
        out_shape=[
            jax.ShapeDtypeStruct((N, IN_DIM), jnp.float32),
            jax.ShapeDtypeStruct((N, IN_DIM), jnp.float32),
        ],
    )(x, Wl, Wr)


def _tc_mid(nd, den8, b1, W3l, W3r):
    D1 = HEADS * HID

    def body(num_ref, den_ref, b1_ref, wl_ref, wr_ref, o_ref):
        num = num_ref[0] + num_ref[1]
        den = den_ref[0] + den_ref[1]
        col = lax.broadcasted_iota(jnp.int32, (HEADS, D1), 1)
        row = lax.broadcasted_iota(jnp.int32, (HEADS, D1), 0)
        bmat = jnp.where(col // HID == row, 1.0, 0.0)
        den_b = jnp.dot(den, bmat, preferred_element_type=jnp.float32)
        h = num / (den_b + 1e-16) + b1_ref[...]
        h = jnp.where(h > 0.0, h, jnp.exp(h) - 1.0)
        o_ref[:, :OUT_DIM] = jnp.dot(
            h, wl_ref[...], preferred_element_type=jnp.float32)
        o_ref[:, OUT_DIM:] = jnp.dot(
            h, wr_ref[...], preferred_element_type=jnp.float32)

    return pl.pallas_call(
        body,
        grid=(_NGA,),
        in_specs=[
            pl.BlockSpec((NC, _RBA, 128), lambda i: (0, i, 0)),
            pl.BlockSpec((NC, _RBA, HEADS), lambda i: (0, i, 0)),
            pl.BlockSpec((1, D1), lambda i: (0, 0)),
            pl.BlockSpec((D1, OUT_DIM), lambda i: (0, 0)),
            pl.BlockSpec((D1, OUT_DIM), lambda i: (0, 0)),
        ],
        out_specs=pl.BlockSpec((_RBA, 2 * OUT_DIM), lambda i: (i, 0)),
        out_shape=jax.ShapeDtypeStruct((NACC, 2 * OUT_DIM), jnp.float32),
    )(nd, den8, b1, W3l, W3r)


def _tc_final(nd, b3):
    def body(nd_ref, b3_ref, o_ref, ls_ref):
        nd0 = nd_ref[0]
        ndb = nd_ref[1]
        num = nd0[:, :OUT_DIM] + ndb[:, :OUT_DIM]
        den = nd0[:, OUT_DIM:OUT_DIM + 1] + ndb[:, OUT_DIM:OUT_DIM + 1]
        o = num / (den + 1e-16) + b3_ref[...]
        o = jnp.where(o > 0.0, o, jnp.exp(o) - 1.0)
        m = jnp.max(o, axis=1, keepdims=True)
        lse = m + jnp.log(jnp.sum(jnp.exp(o - m), axis=1, keepdims=True))
        o_ref[...] = o
        ls_ref[...] = o - lse

    return pl.pallas_call(
        body,
        grid=(_NGA,),
        in_specs=[
            pl.BlockSpec((NC, _RBA, 128), lambda i: (0, i, 0)),
            pl.BlockSpec((1, OUT_DIM), lambda i: (0, 0)),
        ],
        out_specs=[
            pl.BlockSpec((_RBA, OUT_DIM), lambda i: (i, 0)),
            pl.BlockSpec((_RBA, OUT_DIM), lambda i: (i, 0)),
        ],
        out_shape=[
            jax.ShapeDtypeStruct((NACC, OUT_DIM), jnp.float32),
            jax.ShapeDtypeStruct((NACC, OUT_DIM), jnp.float32),
        ],
    )(nd, b3)


def kernel(x, edge_index, W1l, W1r, att1, b1, W3l, W3r, att3, b3):
    eidx = jnp.concatenate(
        [jnp.reshape(edge_index, (-1,)), jnp.zeros((128,), jnp.int32)])
    xl1, xr1 = _tc_in_proj(x, W1l, W1r)
    nd1 = _edge_pass_1(xl1, xr1, eidx, jnp.reshape(att1, (-1,)))
    den8 = jnp.reshape(nd1[:, NACC:, :], (NC, NACC, HEADS))
    hcat = _tc_mid(nd1, den8, jnp.reshape(b1, (1, -1)), W3l, W3r)
    nd2 = _edge_pass_2(hcat, hcat, eidx, jnp.reshape(att3, (-1,)))
    o, ls = _tc_final(nd2, jnp.reshape(b3, (1, -1)))
    return (o[:N], ls[:N])


# speedup vs baseline: 31.4303x; 1.2025x over previous
import functools

import jax
import jax.numpy as jnp
from jax import lax
from jax.experimental import pallas as pl
from jax.experimental.pallas import tpu as pltpu
from jax.experimental.pallas import tpu_sc as plsc

N = 10000
E = 320000
IN_DIM = 128
HID = 16
HEADS = 8
OUT_DIM = 64

NC = 2
NS = 16
NW = NC * NS
EPW = E // NW
KE = 40
KD = 48
NBLK = EPW // KE
NACC = 10240
DROWS = 640
RCH = 8


def _make_edge_pass(D, H, xroff):
    VPH = D // 16 // H
    NROWS = NACC + (DROWS if H > 1 else 0)
    RPT = NROWS // NS
    NCH = RPT // RCH

    mesh = plsc.VectorSubcoreMesh(core_axis_name="c", subcore_axis_name="s")

    scratch = [
        pltpu.VMEM((KE,), jnp.int32),
        pltpu.VMEM((KE,), jnp.int32),
        pltpu.VMEM((KD,), jnp.int32),
        pltpu.VMEM((KD,), jnp.int32),
        pltpu.VMEM((KE, 128), jnp.float32),
        pltpu.VMEM((KE, 128), jnp.float32),
        pltpu.VMEM((KD, 128), jnp.float32),
        pltpu.VMEM((KD, 128), jnp.float32),
        pltpu.VMEM(((2 * KD if H > 1 else KD), 128), jnp.float32),
        pltpu.VMEM((D,), jnp.float32),
        pltpu.VMEM((RCH, 128), jnp.float32),
        pltpu.VMEM_SHARED((NROWS, 128), jnp.float32),
        pltpu.SemaphoreType.DMA,
        pltpu.SemaphoreType.DMA,
        pltpu.SemaphoreType.DMA,
        pltpu.SemaphoreType.DMA,
        pltpu.SemaphoreType.DMA,
        pltpu.SemaphoreType.DMA,
        pltpu.SemaphoreType.DMA,
        pltpu.SemaphoreType.DMA,
    ]
    if H > 1:
        scratch += [
            pltpu.VMEM((2 * KD,), jnp.int32),
        ]
    else:
        scratch += [
            pltpu.VMEM((KD, 128), jnp.float32),
            pltpu.SemaphoreType.DMA,
            pltpu.SemaphoreType.DMA,
        ]

    @functools.partial(
        pl.kernel,
        mesh=mesh,
        out_type=jax.ShapeDtypeStruct((NC, NROWS, 128), jnp.float32),
        scratch_types=scratch,
    )
    def edge_pass(xl_hbm, xr_hbm, eidx_hbm, att_hbm, out_hbm,
                  s0, s1, d0, d1, gl0, gl1, gr0, gr1, wv, attv, rowbuf,
                  acc, ss0, ss1, sd0, sd1, sgl0, sgl1, sgr0, sgr1,
                  *den_scratch):
        if H > 1:
            (cidx,) = den_scratch
            wvr = (wv, wv)
        else:
            wv2, ssc0, ssc1 = den_scratch
            wvr = (wv, wv2)
            SSC = (ssc0, ssc1)
        S = (s0, s1)
        DD = (d0, d1)
        GL = (gl0, gl1)
        GR = (gr0, gr1)
        SS = (ss0, ss1)
        SD = (sd0, sd1)
        SGL = (sgl0, sgl1)
        SGR = (sgr0, sgr1)
        c = lax.axis_index("c")
        s = lax.axis_index("s")
        wid = c * NS + s

        pltpu.sync_copy(att_hbm, attv)

        zerov = jnp.zeros((16,), jnp.float32)

        def _zrow(i, carry):
            for j in range(8):
                rowbuf[i, pl.ds(j * 16, 16)] = zerov
            return carry
        lax.fori_loop(0, RCH, _zrow, 0)

        def _zfire(k, carry):
            r0 = s * RPT + k * RCH
            pltpu.async_copy(rowbuf, acc.at[pl.ds(r0, RCH)], ss0)
            return carry

        def _zdrain(k, carry):
            r0 = s * RPT + k * RCH
            pltpu.make_async_copy(
                rowbuf, acc.at[pl.ds(r0, RCH)], ss0).wait()
            return carry

        lax.fori_loop(0, NCH, _zfire, 0)
        lax.fori_loop(0, NCH, _zdrain, 0)
        plsc.subcore_barrier()

        if H == 1:
            def _ztail(e, carry):
                for j in range(5, 8):
                    wv[e, pl.ds(j * 16, 16)] = zerov
                    wv2[e, pl.ds(j * 16, 16)] = zerov
                return carry
            lax.fori_loop(0, KD, _ztail, 0)
        tails = list(range(KE, KD))
        if H > 1:
            tails += list(range(KD + KE, 2 * KD))
        for e in tails:
            for j in range(8):
                wv[e, pl.ds(j * 16, 16)] = zerov
                if H == 1:
                    wv2[e, pl.ds(j * 16, 16)] = zerov

        attvs = [attv[pl.ds(v * 16, 16)] for v in range(D // 16)]
        lane = lax.iota(jnp.int32, 16)
        lmasks = [lane == h for h in range(H)]
        gdn = lax.GatherDimensionNumbers(
            offset_dims=(), collapsed_slice_dims=(0,), start_index_map=(0,))

        def _perm(u, xor):
            return lax.gather(u, (lane ^ xor)[:, None], gdn, (1,),
                              mode=lax.GatherScatterMode.PROMISE_IN_BOUNDS)

        def _hsum(u):
            for k2 in (8, 4, 2, 1):
                u = u + _perm(u, k2)
            return u

        def _compute_block(xlr, xrr, dstv, wvt):
            def _group(g, jmax):
                if H > 1:
                    dgrp = dstv[pl.ds(g * 16, 16)]
                    og = dgrp & 15
                    oddfv = (og & 1).astype(jnp.float32)
                    vvtfv = lax.shift_right_logical(og, 1).astype(
                        jnp.float32)
                for j in range(jmax):
                    e = g * 16 + j
                    exbs = []
                    for h in range(H):
                        usum = None
                        xls = []
                        for jj in range(VPH):
                            v = h * VPH + jj
                            xlv = xlr[e, pl.ds(v * 16, 16)]
                            xrv = xrr[e, pl.ds(xroff + v * 16, 16)]
                            t = xlv + xrv
                            t = jnp.maximum(t, t * 0.2)
                            u = t * attvs[v]
                            usum = u if usum is None else usum + u
                            xls.append(xlv)
                        exb = jnp.exp(_hsum(usum))
                        exbs.append(exb)
                        for jj in range(VPH):
                            v = h * VPH + jj
                            wvt[e, pl.ds(v * 16, 16)] = xls[jj] * exb
                    if H == 1:
                        wvt[e, pl.ds(D, 16)] = exbs[0]
                    else:
                        exrow = zerov
                        for h in range(H):
                            exrow = jnp.where(lmasks[h], exbs[h], exrow)
                        oddf = oddfv[j]
                        vvtf = vvtfv[j]
                        shifted = exrow + oddf * (_perm(exrow, 8) - exrow)
                        for vv in range(8):
                            m = jnp.maximum(
                                0.0, 1.0 - jnp.abs(vvtf - float(vv)))
                            wvt[KD + e, pl.ds(vv * 16, 16)] = shifted * m

            def _gbody(g, carry):
                _group(g, 16)
                return carry
            lax.fori_loop(0, 2, _gbody, 0)
            _group(2, 8)

        def _fire_idx(b, p):
            base = wid * EPW + b * KE
            pltpu.async_copy(eidx_hbm.at[pl.ds(base, KE)], S[p], SS[p])
            pltpu.async_copy(eidx_hbm.at[pl.ds(E + base, KD)], DD[p], SD[p])

        def _fire_gathers(p):
            pltpu.async_copy(xl_hbm.at[S[p]], GL[p], SGL[p])
            pltpu.async_copy(xr_hbm.at[DD[p]], GR[p], SGR[p])

        def _wait_idx(p):
            pltpu.make_async_copy(
                eidx_hbm.at[pl.ds(0, KE)], S[p], SS[p]).wait()
            pltpu.make_async_copy(
                eidx_hbm.at[pl.ds(0, KD)], DD[p], SD[p]).wait()

        def _wait_gathers(p):
            pltpu.make_async_copy(xl_hbm.at[S[p]], GL[p], SGL[p]).wait()
            pltpu.make_async_copy(xr_hbm.at[DD[p]], GR[p], SGR[p]).wait()

        def _do_block(b, p):
            q = 1 - p
            _wait_idx(q)
            _fire_gathers(q)
            _wait_gathers(p)
            _compute_block(GL[p], GR[p], DD[p], wvr[p])
            if H > 1:
                for g in range(KD // 16):
                    dv = DD[p][pl.ds(g * 16, 16)]
                    cidx[pl.ds(g * 16, 16)] = dv
                    cidx[pl.ds(KD + g * 16, 16)] = (
                        lax.shift_right_logical(dv, 4) + NACC)
                pltpu.sync_copy(wv, acc.at[cidx], add=True)
            else:
                pltpu.async_copy(wvr[p], acc.at[DD[p]], SSC[p], add=True)
            _fire_idx(b + 2, p)

        def _wait_scat(p):
            pltpu.make_async_copy(wvr[p], acc.at[DD[p]], SSC[p]).wait()

        base0 = wid * EPW
        pltpu.sync_copy(eidx_hbm.at[pl.ds(base0, KE)], s0)
        pltpu.sync_copy(eidx_hbm.at[pl.ds(E + base0, KD)], d0)
        _fire_gathers(0)
        _fire_idx(1, 1)

        def _pair(k, carry):
            if H == 1:
                @pl.when(k > 0)
                def _():
                    _wait_scat(0)
                    _wait_scat(1)
            b0 = 2 * k
            _do_block(b0, 0)
            _do_block(b0 + 1, 1)
            return carry

        lax.fori_loop(0, NBLK // 2, _pair, 0)
        _wait_gathers(0)
        _wait_idx(1)
        if H == 1:
            _wait_scat(0)
            _wait_scat(1)
        plsc.subcore_barrier()

        r0 = s * RPT
        pltpu.sync_copy(acc.at[pl.ds(r0, RPT)],
                        out_hbm.at[c, pl.ds(r0, RPT)])

    return edge_pass


_edge_pass_1 = _make_edge_pass(HEADS * HID, HEADS, 0)
_edge_pass_2 = _make_edge_pass(OUT_DIM, 1, OUT_DIM)

_RB = 400
_NG = N // _RB
_RBA = 512
_NGA = NACC // _RBA


def _tc_in_proj(x, Wl, Wr):
    def body(x_ref, wl_ref, wr_ref, xl_ref, xr_ref):
        xb = x_ref[...]
        xl_ref[...] = jnp.dot(xb, wl_ref[...], preferred_element_type=jnp.float32)
        xr_ref[...] = jnp.dot(xb, wr_ref[...], preferred_element_type=jnp.float32)

    return pl.pallas_call(
        body,
        grid=(_NG,),
        in_specs=[
            pl.BlockSpec((_RB, IN_DIM), lambda i: (i, 0)),
            pl.BlockSpec((IN_DIM, IN_DIM), lambda i: (0, 0)),
            pl.BlockSpec((IN_DIM, IN_DIM), lambda i: (0, 0)),
        ],
        out_specs=[
            pl.BlockSpec((_RB, IN_DIM), lambda i: (i, 0)),
            pl.BlockSpec((_RB, IN_DIM), lambda i: (i, 0)),
        ],
        out_shape=[
            jax.ShapeDtypeStruct((N, IN_DIM), jnp.float32),
            jax.ShapeDtypeStruct((N, IN_DIM), jnp.float32),
        ],
    )(x, Wl, Wr)


def _tc_mid(nd, den8, b1, W3l, W3r):
    D1 = HEADS * HID

    def body(num_ref, den_ref, b1_ref, wl_ref, wr_ref, o_ref):
        num = num_ref[0] + num_ref[1]
        den = den_ref[0] + den_ref[1]
        col = lax.broadcasted_iota(jnp.int32, (HEADS, D1), 1)
        row = lax.broadcasted_iota(jnp.int32, (HEADS, D1), 0)
        bmat = jnp.where(col // HID == row, 1.0, 0.0)
        den_b = jnp.dot(den, bmat, preferred_element_type=jnp.float32)
        h = num / (den_b + 1e-16) + b1_ref[...]
        h = jnp.where(h > 0.0, h, jnp.exp(h) - 1.0)
        o_ref[:, :OUT_DIM] = jnp.dot(
            h, wl_ref[...], preferred_element_type=jnp.float32)
        o_ref[:, OUT_DIM:] = jnp.dot(
            h, wr_ref[...], preferred_element_type=jnp.float32)

    return pl.pallas_call(
        body,
        grid=(_NGA,),
        in_specs=[
            pl.BlockSpec((NC, _RBA, 128), lambda i: (0, i, 0)),
            pl.BlockSpec((NC, _RBA, HEADS), lambda i: (0, i, 0)),
            pl.BlockSpec((1, D1), lambda i: (0, 0)),
            pl.BlockSpec((D1, OUT_DIM), lambda i: (0, 0)),
            pl.BlockSpec((D1, OUT_DIM), lambda i: (0, 0)),
        ],
        out_specs=pl.BlockSpec((_RBA, 2 * OUT_DIM), lambda i: (i, 0)),
        out_shape=jax.ShapeDtypeStruct((NACC, 2 * OUT_DIM), jnp.float32),
    )(nd, den8, b1, W3l, W3r)


def _tc_final(nd, b3):
    def body(nd_ref, b3_ref, o_ref, ls_ref):
        nd0 = nd_ref[0]
        ndb = nd_ref[1]
        num = nd0[:, :OUT_DIM] + ndb[:, :OUT_DIM]
        den = nd0[:, OUT_DIM:OUT_DIM + 1] + ndb[:, OUT_DIM:OUT_DIM + 1]
        o = num / (den + 1e-16) + b3_ref[...]
        o = jnp.where(o > 0.0, o, jnp.exp(o) - 1.0)
        m = jnp.max(o, axis=1, keepdims=True)
        lse = m + jnp.log(jnp.sum(jnp.exp(o - m), axis=1, keepdims=True))
        o_ref[...] = o
        ls_ref[...] = o - lse

    return pl.pallas_call(
        body,
        grid=(_NGA,),
        in_specs=[
            pl.BlockSpec((NC, _RBA, 128), lambda i: (0, i, 0)),
            pl.BlockSpec((1, OUT_DIM), lambda i: (0, 0)),
        ],
        out_specs=[
            pl.BlockSpec((_RBA, OUT_DIM), lambda i: (i, 0)),
            pl.BlockSpec((_RBA, OUT_DIM), lambda i: (i, 0)),
        ],
        out_shape=[
            jax.ShapeDtypeStruct((NACC, OUT_DIM), jnp.float32),
            jax.ShapeDtypeStruct((NACC, OUT_DIM), jnp.float32),
        ],
    )(nd, b3)


def kernel(x, edge_index, W1l, W1r, att1, b1, W3l, W3r, att3, b3):
    eidx = jnp.concatenate(
        [jnp.reshape(edge_index, (-1,)), jnp.zeros((128,), jnp.int32)])
    xl1, xr1 = _tc_in_proj(x, W1l, W1r)
    nd1 = _edge_pass_1(xl1, xr1, eidx, jnp.reshape(att1, (-1,)))
    den8 = jnp.reshape(nd1[:, NACC:, :], (NC, NACC, HEADS))
    hcat = _tc_mid(nd1, den8, jnp.reshape(b1, (1, -1)), W3l, W3r)
    nd2 = _edge_pass_2(hcat, hcat, eidx, jnp.reshape(att3, (-1,)))
    o, ls = _tc_final(nd2, jnp.reshape(b3, (1, -1)))
    return (o[:N], ls[:N])


# speedup vs baseline: 33.3711x; 1.0617x over previous
import functools

import jax
import jax.numpy as jnp
from jax import lax
from jax.experimental import pallas as pl
from jax.experimental.pallas import tpu as pltpu
from jax.experimental.pallas import tpu_sc as plsc

N = 10000
E = 320000
IN_DIM = 128
HID = 16
HEADS = 8
OUT_DIM = 64

NC = 2
NS = 16
NW = NC * NS
EPW = E // NW
KE = 40
KD = 48
NBLK = EPW // KE
NACC = 10240
DROWS = 640
RCH = 8


def _make_edge_pass(D, H, xroff):
    VPH = D // 16 // H
    NROWS = NACC + (DROWS if H > 1 else 0)
    RPT = NROWS // NS
    NCH = RPT // RCH

    mesh = plsc.VectorSubcoreMesh(core_axis_name="c", subcore_axis_name="s")

    scratch = [
        pltpu.VMEM((KE,), jnp.int32),
        pltpu.VMEM((KE,), jnp.int32),
        pltpu.VMEM((KD,), jnp.int32),
        pltpu.VMEM((KD,), jnp.int32),
        pltpu.VMEM((KE, 128), jnp.float32),
        pltpu.VMEM((KE, 128), jnp.float32),
        pltpu.VMEM((KD, 128), jnp.float32),
        pltpu.VMEM((KD, 128), jnp.float32),
        pltpu.VMEM(((2 * KD if H > 1 else KD), 128), jnp.float32),
        pltpu.VMEM((D,), jnp.float32),
        pltpu.VMEM((RCH, 128), jnp.float32),
        pltpu.VMEM_SHARED((NROWS, 128), jnp.float32),
        pltpu.SemaphoreType.DMA,
        pltpu.SemaphoreType.DMA,
        pltpu.SemaphoreType.DMA,
        pltpu.SemaphoreType.DMA,
        pltpu.SemaphoreType.DMA,
        pltpu.SemaphoreType.DMA,
        pltpu.SemaphoreType.DMA,
        pltpu.SemaphoreType.DMA,
    ]
    if H > 1:
        scratch += [
            pltpu.VMEM((2 * KD,), jnp.int32),
            pltpu.SemaphoreType.DMA,
        ]
    else:
        scratch += [
            pltpu.VMEM((KD, 128), jnp.float32),
            pltpu.SemaphoreType.DMA,
            pltpu.SemaphoreType.DMA,
        ]

    @functools.partial(
        pl.kernel,
        mesh=mesh,
        out_type=jax.ShapeDtypeStruct((NC, NROWS, 128), jnp.float32),
        scratch_types=scratch,
    )
    def edge_pass(xl_hbm, xr_hbm, eidx_hbm, att_hbm, out_hbm,
                  s0, s1, d0, d1, gl0, gl1, gr0, gr1, wv, attv, rowbuf,
                  acc, ss0, ss1, sd0, sd1, sgl0, sgl1, sgr0, sgr1,
                  *den_scratch):
        if H > 1:
            cidx, ssc = den_scratch
            wvr = (wv, wv)
        else:
            wv2, ssc0, ssc1 = den_scratch
            wvr = (wv, wv2)
            SSC = (ssc0, ssc1)
        S = (s0, s1)
        DD = (d0, d1)
        GL = (gl0, gl1)
        GR = (gr0, gr1)
        SS = (ss0, ss1)
        SD = (sd0, sd1)
        SGL = (sgl0, sgl1)
        SGR = (sgr0, sgr1)
        c = lax.axis_index("c")
        s = lax.axis_index("s")
        wid = c * NS + s

        pltpu.sync_copy(att_hbm, attv)

        zerov = jnp.zeros((16,), jnp.float32)

        def _zrow(i, carry):
            for j in range(8):
                rowbuf[i, pl.ds(j * 16, 16)] = zerov
            return carry
        lax.fori_loop(0, RCH, _zrow, 0)

        def _zfire(k, carry):
            r0 = s * RPT + k * RCH
            pltpu.async_copy(rowbuf, acc.at[pl.ds(r0, RCH)], ss0)
            return carry

        def _zdrain(k, carry):
            r0 = s * RPT + k * RCH
            pltpu.make_async_copy(
                rowbuf, acc.at[pl.ds(r0, RCH)], ss0).wait()
            return carry

        lax.fori_loop(0, NCH, _zfire, 0)
        lax.fori_loop(0, NCH, _zdrain, 0)
        plsc.subcore_barrier()

        if H == 1:
            def _ztail(e, carry):
                for j in range(5, 8):
                    wv[e, pl.ds(j * 16, 16)] = zerov
                    wv2[e, pl.ds(j * 16, 16)] = zerov
                return carry
            lax.fori_loop(0, KD, _ztail, 0)
        tails = list(range(KE, KD))
        if H > 1:
            tails += list(range(KD + KE, 2 * KD))
        for e in tails:
            for j in range(8):
                wv[e, pl.ds(j * 16, 16)] = zerov
                if H == 1:
                    wv2[e, pl.ds(j * 16, 16)] = zerov

        attvs = [attv[pl.ds(v * 16, 16)] for v in range(D // 16)]
        lane = lax.iota(jnp.int32, 16)
        lmasks = [lane == h for h in range(H)]
        gdn = lax.GatherDimensionNumbers(
            offset_dims=(), collapsed_slice_dims=(0,), start_index_map=(0,))

        def _perm(u, xor):
            return lax.gather(u, (lane ^ xor)[:, None], gdn, (1,),
                              mode=lax.GatherScatterMode.PROMISE_IN_BOUNDS)

        def _hsum(u):
            for k2 in (8, 4, 2, 1):
                u = u + _perm(u, k2)
            return u

        def _compute_block(xlr, xrr, dstv, wvt):
            def _group(g, jmax):
                if H > 1:
                    dgrp = dstv[pl.ds(g * 16, 16)]
                    og = dgrp & 15
                    oddfv = (og & 1).astype(jnp.float32)
                    vvtfv = lax.shift_right_logical(og, 1).astype(
                        jnp.float32)
                for j in range(jmax):
                    e = g * 16 + j
                    exbs = []
                    for h in range(H):
                        usum = None
                        xls = []
                        for jj in range(VPH):
                            v = h * VPH + jj
                            xlv = xlr[e, pl.ds(v * 16, 16)]
                            xrv = xrr[e, pl.ds(xroff + v * 16, 16)]
                            t = xlv + xrv
                            t = jnp.maximum(t, t * 0.2)
                            u = t * attvs[v]
                            usum = u if usum is None else usum + u
                            xls.append(xlv)
                        exb = jnp.exp(_hsum(usum))
                        exbs.append(exb)
                        for jj in range(VPH):
                            v = h * VPH + jj
                            wvt[e, pl.ds(v * 16, 16)] = xls[jj] * exb
                    if H == 1:
                        wvt[e, pl.ds(D, 16)] = exbs[0]
                    else:
                        exrow = zerov
                        for h in range(H):
                            exrow = jnp.where(lmasks[h], exbs[h], exrow)
                        oddf = oddfv[j]
                        vvtf = vvtfv[j]
                        shifted = exrow + oddf * (_perm(exrow, 8) - exrow)
                        for vv in range(8):
                            m = jnp.maximum(
                                0.0, 1.0 - jnp.abs(vvtf - float(vv)))
                            wvt[KD + e, pl.ds(vv * 16, 16)] = shifted * m

            def _gbody(g, carry):
                _group(g, 16)
                return carry
            lax.fori_loop(0, 2, _gbody, 0)
            _group(2, 8)

        def _fire_idx(b, p):
            base = wid * EPW + b * KE
            pltpu.async_copy(eidx_hbm.at[pl.ds(base, KE)], S[p], SS[p])
            pltpu.async_copy(eidx_hbm.at[pl.ds(E + base, KD)], DD[p], SD[p])

        def _fire_gathers(p):
            pltpu.async_copy(xl_hbm.at[S[p]], GL[p], SGL[p])
            pltpu.async_copy(xr_hbm.at[DD[p]], GR[p], SGR[p])

        def _wait_idx(p):
            pltpu.make_async_copy(
                eidx_hbm.at[pl.ds(0, KE)], S[p], SS[p]).wait()
            pltpu.make_async_copy(
                eidx_hbm.at[pl.ds(0, KD)], DD[p], SD[p]).wait()

        def _wait_gathers(p):
            pltpu.make_async_copy(xl_hbm.at[S[p]], GL[p], SGL[p]).wait()
            pltpu.make_async_copy(xr_hbm.at[DD[p]], GR[p], SGR[p]).wait()

        def _do_block(b, p, first=False):
            q = 1 - p
            _wait_idx(q)
            _fire_gathers(q)
            _wait_gathers(p)
            if H > 1 and not first:
                @pl.when(b > 0)
                def _():
                    _wait_scat(0)
            _compute_block(GL[p], GR[p], DD[p], wvr[p])
            if H > 1:
                for g in range(KD // 16):
                    dv = DD[p][pl.ds(g * 16, 16)]
                    cidx[pl.ds(g * 16, 16)] = dv
                    cidx[pl.ds(KD + g * 16, 16)] = (
                        lax.shift_right_logical(dv, 4) + NACC)
                pltpu.async_copy(wv, acc.at[cidx], ssc, add=True)
            else:
                pltpu.async_copy(wvr[p], acc.at[DD[p]], SSC[p], add=True)
            _fire_idx(b + 2, p)

        def _wait_scat(p):
            if H > 1:
                pltpu.make_async_copy(wv, acc.at[cidx], ssc).wait()
            else:
                pltpu.make_async_copy(wvr[p], acc.at[DD[p]], SSC[p]).wait()

        base0 = wid * EPW
        pltpu.sync_copy(eidx_hbm.at[pl.ds(base0, KE)], s0)
        pltpu.sync_copy(eidx_hbm.at[pl.ds(E + base0, KD)], d0)
        _fire_gathers(0)
        _fire_idx(1, 1)

        def _pair(k, carry):
            if H == 1:
                @pl.when(k > 0)
                def _():
                    _wait_scat(0)
                    _wait_scat(1)
            b0 = 2 * k
            _do_block(b0, 0)
            _do_block(b0 + 1, 1)
            return carry

        lax.fori_loop(0, NBLK // 2, _pair, 0)
        _wait_gathers(0)
        _wait_idx(1)
        if H == 1:
            _wait_scat(0)
            _wait_scat(1)
        else:
            _wait_scat(0)
        plsc.subcore_barrier()

        r0 = s * RPT
        pltpu.sync_copy(acc.at[pl.ds(r0, RPT)],
                        out_hbm.at[c, pl.ds(r0, RPT)])

    return edge_pass


_edge_pass_1 = _make_edge_pass(HEADS * HID, HEADS, 0)
_edge_pass_2 = _make_edge_pass(OUT_DIM, 1, OUT_DIM)

_RB = 400
_NG = N // _RB
_RBA = 512
_NGA = NACC // _RBA


def _tc_in_proj(x, Wl, Wr):
    def body(x_ref, wl_ref, wr_ref, xl_ref, xr_ref):
        xb = x_ref[...]
        xl_ref[...] = jnp.dot(xb, wl_ref[...], preferred_element_type=jnp.float32)
        xr_ref[...] = jnp.dot(xb, wr_ref[...], preferred_element_type=jnp.float32)

    return pl.pallas_call(
        body,
        grid=(_NG,),
        in_specs=[
            pl.BlockSpec((_RB, IN_DIM), lambda i: (i, 0)),
            pl.BlockSpec((IN_DIM, IN_DIM), lambda i: (0, 0)),
            pl.BlockSpec((IN_DIM, IN_DIM), lambda i: (0, 0)),
        ],
        out_specs=[
            pl.BlockSpec((_RB, IN_DIM), lambda i: (i, 0)),
            pl.BlockSpec((_RB, IN_DIM), lambda i: (i, 0)),
        ],
        out_shape=[
            jax.ShapeDtypeStruct((N, IN_DIM), jnp.float32),
            jax.ShapeDtypeStruct((N, IN_DIM), jnp.float32),
        ],
    )(x, Wl, Wr)


def _tc_mid(nd, den8, b1, W3l, W3r):
    D1 = HEADS * HID

    def body(num_ref, den_ref, b1_ref, wl_ref, wr_ref, o_ref):
        num = num_ref[0] + num_ref[1]
        den = den_ref[0] + den_ref[1]
        col = lax.broadcasted_iota(jnp.int32, (HEADS, D1), 1)
        row = lax.broadcasted_iota(jnp.int32, (HEADS, D1), 0)
        bmat = jnp.where(col // HID == row, 1.0, 0.0)
        den_b = jnp.dot(den, bmat, preferred_element_type=jnp.float32)
        h = num / (den_b + 1e-16) + b1_ref[...]
        h = jnp.where(h > 0.0, h, jnp.exp(h) - 1.0)
        o_ref[:, :OUT_DIM] = jnp.dot(
            h, wl_ref[...], preferred_element_type=jnp.float32)
        o_ref[:, OUT_DIM:] = jnp.dot(
            h, wr_ref[...], preferred_element_type=jnp.float32)

    return pl.pallas_call(
        body,
        grid=(_NGA,),
        in_specs=[
            pl.BlockSpec((NC, _RBA, 128), lambda i: (0, i, 0)),
            pl.BlockSpec((NC, _RBA, HEADS), lambda i: (0, i, 0)),
            pl.BlockSpec((1, D1), lambda i: (0, 0)),
            pl.BlockSpec((D1, OUT_DIM), lambda i: (0, 0)),
            pl.BlockSpec((D1, OUT_DIM), lambda i: (0, 0)),
        ],
        out_specs=pl.BlockSpec((_RBA, 2 * OUT_DIM), lambda i: (i, 0)),
        out_shape=jax.ShapeDtypeStruct((NACC, 2 * OUT_DIM), jnp.float32),
    )(nd, den8, b1, W3l, W3r)


def _tc_final(nd, b3):
    def body(nd_ref, b3_ref, o_ref, ls_ref):
        nd0 = nd_ref[0]
        ndb = nd_ref[1]
        num = nd0[:, :OUT_DIM] + ndb[:, :OUT_DIM]
        den = nd0[:, OUT_DIM:OUT_DIM + 1] + ndb[:, OUT_DIM:OUT_DIM + 1]
        o = num / (den + 1e-16) + b3_ref[...]
        o = jnp.where(o > 0.0, o, jnp.exp(o) - 1.0)
        m = jnp.max(o, axis=1, keepdims=True)
        lse = m + jnp.log(jnp.sum(jnp.exp(o - m), axis=1, keepdims=True))
        o_ref[...] = o
        ls_ref[...] = o - lse

    return pl.pallas_call(
        body,
        grid=(_NGA,),
        in_specs=[
            pl.BlockSpec((NC, _RBA, 128), lambda i: (0, i, 0)),
            pl.BlockSpec((1, OUT_DIM), lambda i: (0, 0)),
        ],
        out_specs=[
            pl.BlockSpec((_RBA, OUT_DIM), lambda i: (i, 0)),
            pl.BlockSpec((_RBA, OUT_DIM), lambda i: (i, 0)),
        ],
        out_shape=[
            jax.ShapeDtypeStruct((NACC, OUT_DIM), jnp.float32),
            jax.ShapeDtypeStruct((NACC, OUT_DIM), jnp.float32),
        ],
    )(nd, b3)


def kernel(x, edge_index, W1l, W1r, att1, b1, W3l, W3r, att3, b3):
    eidx = jnp.concatenate(
        [jnp.reshape(edge_index, (-1,)), jnp.zeros((128,), jnp.int32)])
    xl1, xr1 = _tc_in_proj(x, W1l, W1r)
    nd1 = _edge_pass_1(xl1, xr1, eidx, jnp.reshape(att1, (-1,)))
    den8 = jnp.reshape(nd1[:, NACC:, :], (NC, NACC, HEADS))
    hcat = _tc_mid(nd1, den8, jnp.reshape(b1, (1, -1)), W3l, W3r)
    nd2 = _edge_pass_2(hcat, hcat, eidx, jnp.reshape(att3, (-1,)))
    o, ls = _tc_final(nd2, jnp.reshape(b3, (1, -1)))
    return (o[:N], ls[:N])
